# baseline (speedup 1.0000x reference)
import functools

import jax
import jax.numpy as jnp
from jax import lax
from jax.experimental import pallas as pl
from jax.experimental.pallas import tpu as pltpu
from jax.experimental.pallas import tpu_sc as plsc

H = 64
DN = 128
DE = 16
RO = 1024
PH = 512
LAT = 128
NCLS = 100
B = 256
N = 8192
E = 16384
G = 3
NT = G * N
ET = G * E

SC_CORES = 2
SC_SUB = 16
NW = SC_CORES * SC_SUB
CH = 128
RW = E // NW
KC = RW // CH
NZ = N // SC_SUB

@functools.cache
def _sc_kernels():
    mesh = plsc.VectorSubcoreMesh(
        core_axis_name="c", subcore_axis_name="s",
        num_cores=SC_CORES, num_subcores=SC_SUB)

    @functools.partial(
        pl.kernel,
        out_type=jax.ShapeDtypeStruct((E, H), jnp.float32),
        mesh=mesh,
        compiler_params=pltpu.CompilerParams(use_tc_tiling_on_sc=False),
        scratch_types=[
            pltpu.VMEM((KC, CH), jnp.int32),
            pltpu.VMEM((RW, H), jnp.float32),
            pltpu.SemaphoreType.DMA,
        ],
    )
    def sc_gather(table_hbm, idx_hbm, out_hbm, idx_v, rows_v, sem):
        c = lax.axis_index("c")
        s = lax.axis_index("s")
        wid = s * SC_CORES + c
        pltpu.sync_copy(idx_hbm.at[wid], idx_v)
        descs = []
        for j in range(KC):
            descs.append(pltpu.async_copy(
                table_hbm.at[idx_v.at[j]], rows_v.at[pl.ds(j * CH, CH)], sem))
        for d in descs:
            d.wait()
        pltpu.sync_copy(rows_v, out_hbm.at[pl.ds(wid * RW, RW)])

    @functools.partial(
        pl.kernel,
        out_type=jax.ShapeDtypeStruct((SC_CORES * N, H), jnp.float32),
        mesh=mesh,
        compiler_params=pltpu.CompilerParams(use_tc_tiling_on_sc=False),
        scratch_types=[
            pltpu.VMEM((KC, CH), jnp.int32),
            pltpu.VMEM((2, CH, H), jnp.float32),
            pltpu.VMEM_SHARED((N, H), jnp.float32),
            pltpu.SemaphoreType.DMA,
            pltpu.SemaphoreType.DMA,
        ],
    )
    def sc_scatter(m_hbm, idx_hbm, zeros_hbm, out_hbm, idx_v, m_v, acc_sh,
                   sem0, sem1):
        c = lax.axis_index("c")
        s = lax.axis_index("s")
        zbase = s * NZ
        pltpu.sync_copy(zeros_hbm.at[pl.ds(zbase, NZ)],
                        acc_sh.at[pl.ds(zbase, NZ)])
        ebase = c * (E // SC_CORES) + s * RW
        pltpu.sync_copy(idx_hbm.at[c * SC_SUB + s], idx_v)
        plsc.subcore_barrier()
        sems = (sem0, sem1)
        descs = [None, None]
        descs[0] = pltpu.async_copy(
            m_hbm.at[pl.ds(ebase, CH)], m_v.at[0], sems[0])
        for j in range(KC):
            cur = j % 2
            nxt = (j + 1) % 2
            if j + 1 < KC:
                descs[nxt] = pltpu.async_copy(
                    m_hbm.at[pl.ds(ebase + (j + 1) * CH, CH)],
                    m_v.at[nxt], sems[nxt])
            descs[cur].wait()
            pltpu.sync_copy(m_v.at[cur], acc_sh.at[idx_v.at[j]], add=True)
        plsc.subcore_barrier()
        pltpu.sync_copy(acc_sh.at[pl.ds(zbase, NZ)],
                        out_hbm.at[pl.ds(c * N + zbase, NZ)])

    return sc_gather, sc_scatter


def _sc_gather(table, idx2d):
    return _sc_kernels()[0](table, idx2d)


def _sc_scatter(m, idx2d, zeros):
    return _sc_kernels()[1](m, idx2d, zeros)


def _proj_body(x_ref, w_ref, b_ref, o_ref):
    o_ref[...] = jnp.maximum(
        jnp.dot(x_ref[...], w_ref[...], preferred_element_type=jnp.float32)
        + b_ref[...], 0.0)


_proj = pl.pallas_call(
    _proj_body,
    grid=(NT // 2048,),
    in_specs=[pl.BlockSpec((2048, DN), lambda i: (i, 0)),
              pl.BlockSpec((DN, H), lambda i: (0, 0)),
              pl.BlockSpec((1, H), lambda i: (0, 0))],
    out_specs=pl.BlockSpec((2048, H), lambda i: (i, 0)),
    out_shape=jax.ShapeDtypeStruct((NT, H), jnp.float32),
)


def _msg_body(hs_ref, ea_ref, w3_ref, bm_ref, o_ref):
    hs = hs_ref[...]
    acc = jnp.dot(hs, bm_ref[...], preferred_element_type=jnp.float32)
    for d in range(DE):
        acc = acc + ea_ref[:, d:d + 1] * jnp.dot(
            hs, w3_ref[pl.ds(d * H, H), :], preferred_element_type=jnp.float32)
    o_ref[...] = acc


_msg = pl.pallas_call(
    _msg_body,
    grid=(E // 2048,),
    in_specs=[pl.BlockSpec((2048, H), lambda i: (i, 0)),
              pl.BlockSpec((2048, DE), lambda i: (i, 0)),
              pl.BlockSpec((DE * H, H), lambda i: (0, 0)),
              pl.BlockSpec((H, H), lambda i: (0, 0))],
    out_specs=pl.BlockSpec((2048, H), lambda i: (i, 0)),
    out_shape=jax.ShapeDtypeStruct((E, H), jnp.float32),
)


def _gru_body(p0_ref, p1_ref, h_ref, cb_ref, wr, wz, wn, ur, uz, un,
              br, bz, bi_n, bh_n, o_ref):
    a = jnp.maximum(p0_ref[...] + p1_ref[...] + cb_ref[...], 0.0)
    h = h_ref[...]
    dot = lambda x, w: jnp.dot(x, w[...], preferred_element_type=jnp.float32)
    r = jax.nn.sigmoid(dot(a, wr) + dot(h, ur) + br[...])
    z = jax.nn.sigmoid(dot(a, wz) + dot(h, uz) + bz[...])
    n = jnp.tanh(dot(a, wn) + bi_n[...] + r * (dot(h, un) + bh_n[...]))
    o_ref[...] = (1.0 - z) * n + z * h


_gru = pl.pallas_call(
    _gru_body,
    grid=(N // 2048,),
    in_specs=[pl.BlockSpec((2048, H), lambda i: (i, 0)),
              pl.BlockSpec((2048, H), lambda i: (i, 0)),
              pl.BlockSpec((2048, H), lambda i: (i, 0)),
              pl.BlockSpec((1, H), lambda i: (0, 0))]
             + [pl.BlockSpec((H, H), lambda i: (0, 0))] * 6
             + [pl.BlockSpec((1, H), lambda i: (0, 0))] * 4,
    out_specs=pl.BlockSpec((2048, H), lambda i: (i, 0)),
    out_shape=jax.ShapeDtypeStruct((N, H), jnp.float32),
)


CHK = 1024
NCH = N // CHK


def _s2s_body(x0_ref, h_ref, gidc_ref, wihq, wihr1, wihr2, whh, bb,
              spq, spr1, spr2, spb, spa, out_ref, mask_ref, ex_ref):
    dot = lambda x, w: jnp.dot(x, w, preferred_element_type=jnp.float32)
    iota_cb = lax.broadcasted_iota(jnp.int32, (CHK, B), 1)
    cdims = (((0,), (0,)), ((), ()))
    bdims = (((1,), (1,)), ((), ()))

    def build(ci, z):
        sl = pl.ds(ci * CHK, CHK)
        mask_ref[sl, :] = (gidc_ref[0, sl, :] == iota_cb).astype(jnp.float32)
        return z

    lax.fori_loop(0, NCH, build, 0)

    q = jnp.zeros((B, 2 * H), jnp.float32)
    r1 = jnp.zeros((B, H), jnp.float32)
    r2 = jnp.zeros((B, H), jnp.float32)
    hl = jnp.zeros((B, 2 * H), jnp.float32)
    cl = jnp.zeros((B, 2 * H), jnp.float32)
    for _ in range(3):
        g = (dot(q, wihq[...]) + dot(r1, wihr1[...]) + dot(r2, wihr2[...])
             + dot(hl, whh[...]) + bb[...])
        i = jax.nn.sigmoid(g[:, 0:128])
        f = jax.nn.sigmoid(g[:, 128:256])
        gg = jnp.tanh(g[:, 256:384])
        o = jax.nn.sigmoid(g[:, 384:512])
        cl = f * cl + i * gg
        hl = o * jnp.tanh(cl)
        q = hl
        q1 = q[:, 0:H]
        q2 = q[:, H:2 * H]

        def pass_a(ci, den):
            sl = pl.ds(ci * CHK, CHK)
            mc = mask_ref[sl, :]
            qg1 = dot(mc, q1)
            qg2 = dot(mc, q2)
            esc = (jnp.sum(x0_ref[0, sl, :] * qg1, axis=1, keepdims=True)
                   + jnp.sum(h_ref[0, sl, :] * qg2, axis=1, keepdims=True))
            ex = jnp.exp(esc)
            ex_ref[sl, :] = ex
            return den + lax.dot_general(
                ex, mc, cdims, preferred_element_type=jnp.float32)

        den = lax.fori_loop(0, NCH, pass_a, jnp.zeros((1, B), jnp.float32))

        def pass_b(ci, carry):
            r1a, r2a = carry
            sl = pl.ds(ci * CHK, CHK)
            mc = mask_ref[sl, :]
            den_n = lax.dot_general(mc, den, bdims,
                                    preferred_element_type=jnp.float32)
            alpha = ex_ref[sl, :] / den_n
            r1a = r1a + lax.dot_general(mc, x0_ref[0, sl, :] * alpha, cdims,
                                        preferred_element_type=jnp.float32)
            r2a = r2a + lax.dot_general(mc, h_ref[0, sl, :] * alpha, cdims,
                                        preferred_element_type=jnp.float32)
            return (r1a, r2a)

        r1, r2 = lax.fori_loop(0, NCH, pass_b,
                               (jnp.zeros((B, H), jnp.float32),
                                jnp.zeros((B, H), jnp.float32)))
    out = (dot(q, spq[...]) + dot(r1, spr1[...]) + dot(r2, spr2[...])
           + spb[...])
    a = spa[0, 0]
    out_ref[0] = jnp.where(out >= 0, out, a * out)


_s2s = pl.pallas_call(
    _s2s_body,
    grid=(G,),
    scratch_shapes=[pltpu.VMEM((N, B), jnp.float32),
                    pltpu.VMEM((N, 1), jnp.float32)],
    in_specs=[pl.BlockSpec((1, N, H), lambda g: (g, 0, 0)),
              pl.BlockSpec((1, N, H), lambda g: (g, 0, 0)),
              pl.BlockSpec((1, N, 1), lambda g: (g, 0, 0)),
              pl.BlockSpec((2 * H, 512), lambda g: (0, 0)),
              pl.BlockSpec((H, 512), lambda g: (0, 0)),
              pl.BlockSpec((H, 512), lambda g: (0, 0)),
              pl.BlockSpec((2 * H, 512), lambda g: (0, 0)),
              pl.BlockSpec((1, 512), lambda g: (0, 0)),
              pl.BlockSpec((2 * H, RO), lambda g: (0, 0)),
              pl.BlockSpec((H, RO), lambda g: (0, 0)),
              pl.BlockSpec((H, RO), lambda g: (0, 0)),
              pl.BlockSpec((1, RO), lambda g: (0, 0)),
              pl.BlockSpec((1, 1), lambda g: (0, 0))],
    out_specs=pl.BlockSpec((1, B, RO), lambda g: (g, 0, 0)),
    out_shape=jax.ShapeDtypeStruct((G, B, RO), jnp.float32),
)


def _mlp_body(xin, eps, ge, pnv,
              w0, b0, a0, w1, b1, a1, w2, b2, a2, w3, b3,
              dw0a, dw0b, dw0c, db0, da0, dw1, db1, da1,
              dw2, db2, da2, dw3, db3,
              y_ref, mu_ref, lv_ref):
    dot = lambda x, w: jnp.dot(x, w[...], preferred_element_type=jnp.float32)
    prelu = lambda x, a: jnp.where(x >= 0, x, a[0, 0] * x)
    x = prelu(dot(xin[...], w0) + b0[...], a0)
    x = prelu(dot(x, w1) + b1[...], a1)
    x = prelu(dot(x, w2) + b2[...], a2)
    x = dot(x, w3) + b3[...]
    mu = jnp.clip(x[:, 0:LAT], -10.0, 10.0)
    lv = jnp.clip(x[:, LAT:2 * LAT], -10.0, 10.0)
    latent = mu + eps[...] * jnp.exp(0.5 * lv)
    y = prelu(dot(latent, dw0a) + dot(ge[...], dw0b)
              + pnv[0, 0] * dw0c[...] + db0[...], da0)
    y = prelu(dot(y, dw1) + db1[...], da1)
    y = prelu(dot(y, dw2) + db2[...], da2)
    y = dot(y, dw3) + db3[...]
    y_ref[...] = jnp.clip(y, -10.0, 10.0)
    mu_ref[...] = mu
    lv_ref[...] = lv


_EI = RO * G + NCLS + 1


def _full(shape):
    return pl.BlockSpec(shape, lambda: tuple(0 for _ in shape))


_mlp = pl.pallas_call(
    _mlp_body,
    in_specs=[_full((B, _EI)), _full((B, LAT)), _full((B, G * RO)),
              _full((1, 1)),
              _full((_EI, PH)), _full((1, PH)), _full((1, 1)),
              _full((PH, PH)), _full((1, PH)), _full((1, 1)),
              _full((PH, PH)), _full((1, PH)), _full((1, 1)),
              _full((PH, 2 * LAT)), _full((1, 2 * LAT)),
              _full((LAT, PH)), _full((G * RO, PH)), _full((1, PH)),
              _full((1, PH)), _full((1, 1)),
              _full((PH, PH)), _full((1, PH)), _full((1, 1)),
              _full((PH, PH)), _full((1, PH)), _full((1, 1)),
              _full((PH, NCLS)), _full((1, NCLS))],
    out_specs=[_full((B, NCLS)), _full((B, LAT)), _full((B, LAT))],
    out_shape=[jax.ShapeDtypeStruct((B, NCLS), jnp.float32),
               jax.ShapeDtypeStruct((B, LAT), jnp.float32),
               jax.ShapeDtypeStruct((B, LAT), jnp.float32)],
)


def kernel(r1_x, r1_e, r1_src, r1_dst, r1_gid, r2_x, r2_e, r2_src, r2_dst,
           r2_gid, pm_x, pm_e, pm_src, pm_dst, pm_gid, labels,
           pos_neg_sample, params):
    p = params
    f32 = jnp.float32
    x_all = jnp.concatenate([r1_x, r2_x, pm_x], axis=0)
    e_g = [r1_e, r2_e, pm_e]
    src_g = [s.astype(jnp.int32).reshape(NW, KC, CH)
             for s in (r1_src, r2_src, pm_src)]
    dst_g = [s.astype(jnp.int32).reshape(NW, KC, CH)
             for s in (r1_dst, r2_dst, pm_dst)]
    zeros_n = jnp.zeros((N, H), f32)

    x0 = _proj(x_all, p['proj_W'], p['proj_b'].reshape(1, H))

    w3 = p['edge_W'].reshape(DE * H, H)
    bm = p['edge_b'].reshape(H, H)
    wih = p['gru_Wih']
    whh = p['gru_Whh']
    wr, wz, wn = (wih[0:H].T, wih[H:2 * H].T, wih[2 * H:].T)
    ur, uz, un = (whh[0:H].T, whh[H:2 * H].T, whh[2 * H:].T)
    bih = p['gru_bih']
    bhh = p['gru_bhh']
    br = (bih[0:H] + bhh[0:H]).reshape(1, H)
    bz = (bih[H:2 * H] + bhh[H:2 * H]).reshape(1, H)
    bi_n = bih[2 * H:].reshape(1, H)
    bh_n = bhh[2 * H:].reshape(1, H)
    cb = p['conv_b'].reshape(1, H)

    hg = [x0[0:N], x0[N:2 * N], x0[2 * N:3 * N]]
    for _ in range(3):
        for g in range(G):
            hsrc = _sc_gather(hg[g], src_g[g])
            m = _msg(hsrc, e_g[g], w3, bm)
            parts = _sc_scatter(m, dst_g[g], zeros_n)
            hg[g] = _gru(parts[0:N], parts[N:2 * N], hg[g], cb, wr, wz, wn,
                         ur, uz, un, br, bz, bi_n, bh_n)
    h = jnp.concatenate(hg, axis=0)

    gid_all = jnp.stack([r1_gid, r2_gid, pm_gid]).astype(jnp.int32)
    wihT = p['lstm_Wih'].T
    go = _s2s(x0.reshape(G, N, H), h.reshape(G, N, H),
              gid_all.reshape(G, N, 1),
              wihT[0:2 * H], wihT[2 * H:3 * H], wihT[3 * H:4 * H],
              p['lstm_Whh'].T,
              (p['lstm_bih'] + p['lstm_bhh']).reshape(1, 8 * H),
              p['sp_W'][0:2 * H], p['sp_W'][2 * H:3 * H], p['sp_W'][3 * H:],
              p['sp_b'].reshape(1, RO), p['sp_a'].reshape(1, 1))

    ge = jnp.transpose(go, (1, 0, 2)).reshape(B, G * RO)
    pos = jnp.asarray(pos_neg_sample).astype(f32)
    pn = jnp.zeros((B, 1), f32) + pos
    xin = jnp.concatenate([labels, ge, pn], axis=1)
    eps = jax.random.normal(jax.random.key(42), (B, LAT), dtype=f32)
    dw0 = p['dec_W0']
    y, mu, lv = _mlp(
        xin, eps, ge, pos.reshape(1, 1),
        p['enc_W0'], p['enc_b0'].reshape(1, PH), p['enc_a0'].reshape(1, 1),
        p['enc_W1'], p['enc_b1'].reshape(1, PH), p['enc_a1'].reshape(1, 1),
        p['enc_W2'], p['enc_b2'].reshape(1, PH), p['enc_a2'].reshape(1, 1),
        p['enc_W3'], p['enc_b3'].reshape(1, 2 * LAT),
        dw0[0:LAT], dw0[LAT:LAT + G * RO], dw0[LAT + G * RO:],
        p['dec_b0'].reshape(1, PH), p['dec_a0'].reshape(1, 1),
        p['dec_W1'], p['dec_b1'].reshape(1, PH), p['dec_a1'].reshape(1, 1),
        p['dec_W2'], p['dec_b2'].reshape(1, PH), p['dec_a2'].reshape(1, 1),
        p['dec_W3'], p['dec_b3'].reshape(1, NCLS))
    return (y, mu, lv)


# speedup vs baseline: 1.2106x; 1.2106x over previous
import functools

import jax
import jax.numpy as jnp
from jax import lax
from jax.experimental import pallas as pl
from jax.experimental.pallas import tpu as pltpu
from jax.experimental.pallas import tpu_sc as plsc

H = 64
DN = 128
DE = 16
RO = 1024
PH = 512
LAT = 128
NCLS = 100
B = 256
N = 8192
E = 16384
G = 3
NT = G * N
ET = G * E

SC_CORES = 2
SC_SUB = 16
NW = SC_CORES * SC_SUB
CH = 128
RW = ET // NW
KC = RW // CH
NZ = NT // SC_SUB

@functools.cache
def _sc_kernels():
    mesh = plsc.VectorSubcoreMesh(
        core_axis_name="c", subcore_axis_name="s",
        num_cores=SC_CORES, num_subcores=SC_SUB)

    @functools.partial(
        pl.kernel,
        out_type=jax.ShapeDtypeStruct((ET, H), jnp.float32),
        mesh=mesh,
        compiler_params=pltpu.CompilerParams(use_tc_tiling_on_sc=False),
        scratch_types=[
            pltpu.VMEM((KC, CH), jnp.int32),
            pltpu.VMEM((RW, H), jnp.float32),
            pltpu.SemaphoreType.DMA,
        ],
    )
    def sc_gather(table_hbm, idx_hbm, out_hbm, idx_v, rows_v, sem):
        c = lax.axis_index("c")
        s = lax.axis_index("s")
        wid = s * SC_CORES + c
        pltpu.sync_copy(idx_hbm.at[wid], idx_v)
        descs = []
        for j in range(KC):
            descs.append(pltpu.async_copy(
                table_hbm.at[idx_v.at[j]], rows_v.at[pl.ds(j * CH, CH)], sem))
        for d in descs:
            d.wait()
        pltpu.sync_copy(rows_v, out_hbm.at[pl.ds(wid * RW, RW)])

    @functools.partial(
        pl.kernel,
        out_type=jax.ShapeDtypeStruct((SC_CORES * NT, H), jnp.float32),
        mesh=mesh,
        compiler_params=pltpu.CompilerParams(use_tc_tiling_on_sc=False),
        scratch_types=[
            pltpu.VMEM((KC, CH), jnp.int32),
            pltpu.VMEM((2, CH, H), jnp.float32),
            pltpu.VMEM_SHARED((NT, H), jnp.float32),
            pltpu.SemaphoreType.DMA,
            pltpu.SemaphoreType.DMA,
        ],
    )
    def sc_scatter(m_hbm, idx_hbm, zeros_hbm, out_hbm, idx_v, m_v, acc_sh,
                   sem0, sem1):
        c = lax.axis_index("c")
        s = lax.axis_index("s")
        zbase = s * NZ
        pltpu.sync_copy(zeros_hbm.at[pl.ds(zbase, NZ)],
                        acc_sh.at[pl.ds(zbase, NZ)])
        ebase = c * (ET // SC_CORES) + s * RW
        pltpu.sync_copy(idx_hbm.at[c * SC_SUB + s], idx_v)
        plsc.subcore_barrier()
        sems = (sem0, sem1)
        descs = [None, None]
        descs[0] = pltpu.async_copy(
            m_hbm.at[pl.ds(ebase, CH)], m_v.at[0], sems[0])
        for j in range(KC):
            cur = j % 2
            nxt = (j + 1) % 2
            if j + 1 < KC:
                descs[nxt] = pltpu.async_copy(
                    m_hbm.at[pl.ds(ebase + (j + 1) * CH, CH)],
                    m_v.at[nxt], sems[nxt])
            descs[cur].wait()
            pltpu.sync_copy(m_v.at[cur], acc_sh.at[idx_v.at[j]], add=True)
        plsc.subcore_barrier()
        pltpu.sync_copy(acc_sh.at[pl.ds(zbase, NZ)],
                        out_hbm.at[pl.ds(c * NT + zbase, NZ)])

    return sc_gather, sc_scatter


def _sc_gather(table, idx2d):
    return _sc_kernels()[0](table, idx2d)


def _sc_scatter(m, idx2d, zeros):
    return _sc_kernels()[1](m, idx2d, zeros)


def _proj_body(x_ref, w_ref, b_ref, o_ref):
    o_ref[...] = jnp.maximum(
        jnp.dot(x_ref[...], w_ref[...], preferred_element_type=jnp.float32)
        + b_ref[...], 0.0)


_proj = pl.pallas_call(
    _proj_body,
    grid=(NT // 2048,),
    in_specs=[pl.BlockSpec((2048, DN), lambda i: (i, 0)),
              pl.BlockSpec((DN, H), lambda i: (0, 0)),
              pl.BlockSpec((1, H), lambda i: (0, 0))],
    out_specs=pl.BlockSpec((2048, H), lambda i: (i, 0)),
    out_shape=jax.ShapeDtypeStruct((NT, H), jnp.float32),
)


def _msg_body(hs4_ref, ea4_ref, w3bd_ref, bmbd_ref, s_ref, o_ref):
    hs4 = hs4_ref[...]
    ea4 = ea4_ref[...]
    acc = jnp.dot(hs4, bmbd_ref[...], preferred_element_type=jnp.float32)
    for d in range(DE):
        scale = jnp.dot(ea4, s_ref[pl.ds(d * H, H), :],
                        preferred_element_type=jnp.float32)
        acc = acc + scale * jnp.dot(
            hs4, w3bd_ref[pl.ds(d * 4 * H, 4 * H), :],
            preferred_element_type=jnp.float32)
    o_ref[...] = acc


_msg = pl.pallas_call(
    _msg_body,
    grid=(ET // 2048,),
    in_specs=[pl.BlockSpec((512, 4 * H), lambda i: (i, 0)),
              pl.BlockSpec((512, 4 * DE), lambda i: (i, 0)),
              pl.BlockSpec((DE * 4 * H, 4 * H), lambda i: (0, 0)),
              pl.BlockSpec((4 * H, 4 * H), lambda i: (0, 0)),
              pl.BlockSpec((DE * H, 4 * H), lambda i: (0, 0))],
    out_specs=pl.BlockSpec((512, 4 * H), lambda i: (i, 0)),
    out_shape=jax.ShapeDtypeStruct((ET // 4, 4 * H), jnp.float32),
)


def _gru_body(p0_ref, p1_ref, h_ref, cb_ref, wr, wz, wn, ur, uz, un,
              br, bz, bi_n, bh_n, o_ref):
    a = jnp.maximum(p0_ref[...] + p1_ref[...] + cb_ref[...], 0.0)
    h = h_ref[...]
    dot = lambda x, w: jnp.dot(x, w[...], preferred_element_type=jnp.float32)
    r = jax.nn.sigmoid(dot(a, wr) + dot(h, ur) + br[...])
    z = jax.nn.sigmoid(dot(a, wz) + dot(h, uz) + bz[...])
    n = jnp.tanh(dot(a, wn) + bi_n[...] + r * (dot(h, un) + bh_n[...]))
    o_ref[...] = (1.0 - z) * n + z * h


_gru = pl.pallas_call(
    _gru_body,
    grid=(NT // 2048,),
    in_specs=[pl.BlockSpec((2048, H), lambda i: (i, 0)),
              pl.BlockSpec((2048, H), lambda i: (i, 0)),
              pl.BlockSpec((2048, H), lambda i: (i, 0)),
              pl.BlockSpec((1, H), lambda i: (0, 0))]
             + [pl.BlockSpec((H, H), lambda i: (0, 0))] * 6
             + [pl.BlockSpec((1, H), lambda i: (0, 0))] * 4,
    out_specs=pl.BlockSpec((2048, H), lambda i: (i, 0)),
    out_shape=jax.ShapeDtypeStruct((NT, H), jnp.float32),
)


CHK = 1024
NCH = N // CHK


def _s2s_body(x0_ref, h_ref, gidc_ref, wihq, wihr1, wihr2, whh, bb,
              spq, spr1, spr2, spb, spa, out_ref, mask_ref, ex_ref):
    dot = lambda x, w: jnp.dot(x, w, preferred_element_type=jnp.float32)
    iota_cb = lax.broadcasted_iota(jnp.int32, (CHK, B), 1)
    cdims = (((0,), (0,)), ((), ()))
    bdims = (((1,), (1,)), ((), ()))

    def build(ci, z):
        sl = pl.ds(ci * CHK, CHK)
        mask_ref[sl, :] = (gidc_ref[0, sl, :] == iota_cb).astype(jnp.float32)
        return z

    lax.fori_loop(0, NCH, build, 0)

    q = jnp.zeros((B, 2 * H), jnp.float32)
    r1 = jnp.zeros((B, H), jnp.float32)
    r2 = jnp.zeros((B, H), jnp.float32)
    hl = jnp.zeros((B, 2 * H), jnp.float32)
    cl = jnp.zeros((B, 2 * H), jnp.float32)
    for _ in range(3):
        g = (dot(q, wihq[...]) + dot(r1, wihr1[...]) + dot(r2, wihr2[...])
             + dot(hl, whh[...]) + bb[...])
        i = jax.nn.sigmoid(g[:, 0:128])
        f = jax.nn.sigmoid(g[:, 128:256])
        gg = jnp.tanh(g[:, 256:384])
        o = jax.nn.sigmoid(g[:, 384:512])
        cl = f * cl + i * gg
        hl = o * jnp.tanh(cl)
        q = hl
        q1 = q[:, 0:H]
        q2 = q[:, H:2 * H]

        def pass_a(ci, den):
            sl = pl.ds(ci * CHK, CHK)
            mc = mask_ref[sl, :]
            qg1 = dot(mc, q1)
            qg2 = dot(mc, q2)
            esc = (jnp.sum(x0_ref[0, sl, :] * qg1, axis=1, keepdims=True)
                   + jnp.sum(h_ref[0, sl, :] * qg2, axis=1, keepdims=True))
            ex = jnp.exp(esc)
            ex_ref[sl, :] = ex
            return den + lax.dot_general(
                ex, mc, cdims, preferred_element_type=jnp.float32)

        den = lax.fori_loop(0, NCH, pass_a, jnp.zeros((1, B), jnp.float32))

        def pass_b(ci, carry):
            r1a, r2a = carry
            sl = pl.ds(ci * CHK, CHK)
            mc = mask_ref[sl, :]
            den_n = lax.dot_general(mc, den, bdims,
                                    preferred_element_type=jnp.float32)
            alpha = ex_ref[sl, :] / den_n
            r1a = r1a + lax.dot_general(mc, x0_ref[0, sl, :] * alpha, cdims,
                                        preferred_element_type=jnp.float32)
            r2a = r2a + lax.dot_general(mc, h_ref[0, sl, :] * alpha, cdims,
                                        preferred_element_type=jnp.float32)
            return (r1a, r2a)

        r1, r2 = lax.fori_loop(0, NCH, pass_b,
                               (jnp.zeros((B, H), jnp.float32),
                                jnp.zeros((B, H), jnp.float32)))
    out = (dot(q, spq[...]) + dot(r1, spr1[...]) + dot(r2, spr2[...])
           + spb[...])
    a = spa[0, 0]
    out_ref[0] = jnp.where(out >= 0, out, a * out)


_s2s = pl.pallas_call(
    _s2s_body,
    grid=(G,),
    scratch_shapes=[pltpu.VMEM((N, B), jnp.float32),
                    pltpu.VMEM((N, 1), jnp.float32)],
    in_specs=[pl.BlockSpec((1, N, H), lambda g: (g, 0, 0)),
              pl.BlockSpec((1, N, H), lambda g: (g, 0, 0)),
              pl.BlockSpec((1, N, 1), lambda g: (g, 0, 0)),
              pl.BlockSpec((2 * H, 512), lambda g: (0, 0)),
              pl.BlockSpec((H, 512), lambda g: (0, 0)),
              pl.BlockSpec((H, 512), lambda g: (0, 0)),
              pl.BlockSpec((2 * H, 512), lambda g: (0, 0)),
              pl.BlockSpec((1, 512), lambda g: (0, 0)),
              pl.BlockSpec((2 * H, RO), lambda g: (0, 0)),
              pl.BlockSpec((H, RO), lambda g: (0, 0)),
              pl.BlockSpec((H, RO), lambda g: (0, 0)),
              pl.BlockSpec((1, RO), lambda g: (0, 0)),
              pl.BlockSpec((1, 1), lambda g: (0, 0))],
    out_specs=pl.BlockSpec((1, B, RO), lambda g: (g, 0, 0)),
    out_shape=jax.ShapeDtypeStruct((G, B, RO), jnp.float32),
)


def _mlp_body(xin, eps, ge, pnv,
              w0, b0, a0, w1, b1, a1, w2, b2, a2, w3, b3,
              dw0a, dw0b, dw0c, db0, da0, dw1, db1, da1,
              dw2, db2, da2, dw3, db3,
              y_ref, mu_ref, lv_ref):
    dot = lambda x, w: jnp.dot(x, w[...], preferred_element_type=jnp.float32)
    prelu = lambda x, a: jnp.where(x >= 0, x, a[0, 0] * x)
    x = prelu(dot(xin[...], w0) + b0[...], a0)
    x = prelu(dot(x, w1) + b1[...], a1)
    x = prelu(dot(x, w2) + b2[...], a2)
    x = dot(x, w3) + b3[...]
    mu = jnp.clip(x[:, 0:LAT], -10.0, 10.0)
    lv = jnp.clip(x[:, LAT:2 * LAT], -10.0, 10.0)
    latent = mu + eps[...] * jnp.exp(0.5 * lv)
    y = prelu(dot(latent, dw0a) + dot(ge[...], dw0b)
              + pnv[0, 0] * dw0c[...] + db0[...], da0)
    y = prelu(dot(y, dw1) + db1[...], da1)
    y = prelu(dot(y, dw2) + db2[...], da2)
    y = dot(y, dw3) + db3[...]
    y_ref[...] = jnp.clip(y, -10.0, 10.0)
    mu_ref[...] = mu
    lv_ref[...] = lv


_EI = RO * G + NCLS + 1


def _full(shape):
    return pl.BlockSpec(shape, lambda: tuple(0 for _ in shape))


_mlp = pl.pallas_call(
    _mlp_body,
    in_specs=[_full((B, _EI)), _full((B, LAT)), _full((B, G * RO)),
              _full((1, 1)),
              _full((_EI, PH)), _full((1, PH)), _full((1, 1)),
              _full((PH, PH)), _full((1, PH)), _full((1, 1)),
              _full((PH, PH)), _full((1, PH)), _full((1, 1)),
              _full((PH, 2 * LAT)), _full((1, 2 * LAT)),
              _full((LAT, PH)), _full((G * RO, PH)), _full((1, PH)),
              _full((1, PH)), _full((1, 1)),
              _full((PH, PH)), _full((1, PH)), _full((1, 1)),
              _full((PH, PH)), _full((1, PH)), _full((1, 1)),
              _full((PH, NCLS)), _full((1, NCLS))],
    out_specs=[_full((B, NCLS)), _full((B, LAT)), _full((B, LAT))],
    out_shape=[jax.ShapeDtypeStruct((B, NCLS), jnp.float32),
               jax.ShapeDtypeStruct((B, LAT), jnp.float32),
               jax.ShapeDtypeStruct((B, LAT), jnp.float32)],
)


def kernel(r1_x, r1_e, r1_src, r1_dst, r1_gid, r2_x, r2_e, r2_src, r2_dst,
           r2_gid, pm_x, pm_e, pm_src, pm_dst, pm_gid, labels,
           pos_neg_sample, params):
    p = params
    f32 = jnp.float32
    x_all = jnp.concatenate([r1_x, r2_x, pm_x], axis=0)
    e_all = jnp.concatenate([r1_e, r2_e, pm_e], axis=0)
    src_all = jnp.concatenate(
        [r1_src, r2_src + N, pm_src + 2 * N]).astype(jnp.int32)
    dst_all = jnp.concatenate(
        [r1_dst, r2_dst + N, pm_dst + 2 * N]).astype(jnp.int32)
    src2d = src_all.reshape(NW, KC, CH)
    dst2d = dst_all.reshape(NW, KC, CH)
    zeros_nt = jnp.zeros((NT, H), f32)

    x0 = _proj(x_all, p['proj_W'], p['proj_b'].reshape(1, H))

    eye4 = jnp.eye(4, dtype=f32)
    w3s = p['edge_W'].reshape(DE, H, H)
    w3bd = jnp.concatenate(
        [jnp.kron(eye4, w3s[d]) for d in range(DE)], axis=0)
    bmbd = jnp.kron(eye4, p['edge_b'].reshape(H, H))
    kk = jnp.arange(4 * DE)[None, :, None]
    blk = jnp.arange(4 * H)[None, None, :] // H
    dsel = jnp.arange(DE)[:, None, None]
    smat = (kk == blk * DE + dsel).astype(f32).reshape(DE * 4 * DE, 4 * H)
    wih = p['gru_Wih']
    whh = p['gru_Whh']
    wr, wz, wn = (wih[0:H].T, wih[H:2 * H].T, wih[2 * H:].T)
    ur, uz, un = (whh[0:H].T, whh[H:2 * H].T, whh[2 * H:].T)
    bih = p['gru_bih']
    bhh = p['gru_bhh']
    br = (bih[0:H] + bhh[0:H]).reshape(1, H)
    bz = (bih[H:2 * H] + bhh[H:2 * H]).reshape(1, H)
    bi_n = bih[2 * H:].reshape(1, H)
    bh_n = bhh[2 * H:].reshape(1, H)
    cb = p['conv_b'].reshape(1, H)

    h = x0
    for _ in range(3):
        hsrc = _sc_gather(h, src2d)
        m4 = _msg(hsrc.reshape(ET // 4, 4 * H), e_all.reshape(ET // 4, 4 * DE),
                  w3bd, bmbd, smat)
        parts = _sc_scatter(m4.reshape(ET, H), dst2d, zeros_nt)
        h = _gru(parts[0:NT], parts[NT:2 * NT], h, cb, wr, wz, wn,
                 ur, uz, un, br, bz, bi_n, bh_n)

    gid_all = jnp.stack([r1_gid, r2_gid, pm_gid]).astype(jnp.int32)
    wihT = p['lstm_Wih'].T
    go = _s2s(x0.reshape(G, N, H), h.reshape(G, N, H),
              gid_all.reshape(G, N, 1),
              wihT[0:2 * H], wihT[2 * H:3 * H], wihT[3 * H:4 * H],
              p['lstm_Whh'].T,
              (p['lstm_bih'] + p['lstm_bhh']).reshape(1, 8 * H),
              p['sp_W'][0:2 * H], p['sp_W'][2 * H:3 * H], p['sp_W'][3 * H:],
              p['sp_b'].reshape(1, RO), p['sp_a'].reshape(1, 1))

    ge = jnp.transpose(go, (1, 0, 2)).reshape(B, G * RO)
    pos = jnp.asarray(pos_neg_sample).astype(f32)
    pn = jnp.zeros((B, 1), f32) + pos
    xin = jnp.concatenate([labels, ge, pn], axis=1)
    eps = jax.random.normal(jax.random.key(42), (B, LAT), dtype=f32)
    dw0 = p['dec_W0']
    y, mu, lv = _mlp(
        xin, eps, ge, pos.reshape(1, 1),
        p['enc_W0'], p['enc_b0'].reshape(1, PH), p['enc_a0'].reshape(1, 1),
        p['enc_W1'], p['enc_b1'].reshape(1, PH), p['enc_a1'].reshape(1, 1),
        p['enc_W2'], p['enc_b2'].reshape(1, PH), p['enc_a2'].reshape(1, 1),
        p['enc_W3'], p['enc_b3'].reshape(1, 2 * LAT),
        dw0[0:LAT], dw0[LAT:LAT + G * RO], dw0[LAT + G * RO:],
        p['dec_b0'].reshape(1, PH), p['dec_a0'].reshape(1, 1),
        p['dec_W1'], p['dec_b1'].reshape(1, PH), p['dec_a1'].reshape(1, 1),
        p['dec_W2'], p['dec_b2'].reshape(1, PH), p['dec_a2'].reshape(1, 1),
        p['dec_W3'], p['dec_b3'].reshape(1, NCLS))
    return (y, mu, lv)


# speedup vs baseline: 1.2225x; 1.0098x over previous
import functools

import jax
import jax.numpy as jnp
from jax import lax
from jax.experimental import pallas as pl
from jax.experimental.pallas import tpu as pltpu
from jax.experimental.pallas import tpu_sc as plsc

H = 64
DN = 128
DE = 16
RO = 1024
PH = 512
LAT = 128
NCLS = 100
B = 256
N = 8192
E = 16384
G = 3
NT = G * N
ET = G * E

SC_CORES = 2
SC_SUB = 16
NW = SC_CORES * SC_SUB
CH = 128
RW = ET // NW
KC = RW // CH
NZ = NT // SC_SUB

@functools.cache
def _sc_kernels():
    mesh = plsc.VectorSubcoreMesh(
        core_axis_name="c", subcore_axis_name="s",
        num_cores=SC_CORES, num_subcores=SC_SUB)

    @functools.partial(
        pl.kernel,
        out_type=jax.ShapeDtypeStruct((ET, H), jnp.float32),
        mesh=mesh,
        compiler_params=pltpu.CompilerParams(use_tc_tiling_on_sc=False),
        scratch_types=[
            pltpu.VMEM((KC, CH), jnp.int32),
            pltpu.VMEM((RW, H), jnp.float32),
            pltpu.SemaphoreType.DMA,
        ],
    )
    def sc_gather(table_hbm, idx_hbm, out_hbm, idx_v, rows_v, sem):
        c = lax.axis_index("c")
        s = lax.axis_index("s")
        wid = s * SC_CORES + c
        pltpu.sync_copy(idx_hbm.at[wid], idx_v)
        descs = []
        for j in range(KC):
            descs.append(pltpu.async_copy(
                table_hbm.at[idx_v.at[j]], rows_v.at[pl.ds(j * CH, CH)], sem))
        for d in descs:
            d.wait()
        pltpu.sync_copy(rows_v, out_hbm.at[pl.ds(wid * RW, RW)])

    @functools.partial(
        pl.kernel,
        out_type=jax.ShapeDtypeStruct((SC_CORES * NT, H), jnp.float32),
        mesh=mesh,
        compiler_params=pltpu.CompilerParams(use_tc_tiling_on_sc=False),
        scratch_types=[
            pltpu.VMEM((KC, CH), jnp.int32),
            pltpu.VMEM((2, CH, H), jnp.float32),
            pltpu.VMEM_SHARED((NT, H), jnp.float32),
            pltpu.SemaphoreType.DMA,
            pltpu.SemaphoreType.DMA,
        ],
    )
    def sc_scatter(m_hbm, idx_hbm, zeros_hbm, out_hbm, idx_v, m_v, acc_sh,
                   sem0, sem1):
        c = lax.axis_index("c")
        s = lax.axis_index("s")
        zbase = s * NZ
        pltpu.sync_copy(zeros_hbm.at[pl.ds(zbase, NZ)],
                        acc_sh.at[pl.ds(zbase, NZ)])
        ebase = c * (ET // SC_CORES) + s * RW
        pltpu.sync_copy(idx_hbm.at[c * SC_SUB + s], idx_v)
        plsc.subcore_barrier()
        sems = (sem0, sem1)
        descs = [None, None]
        descs[0] = pltpu.async_copy(
            m_hbm.at[pl.ds(ebase, CH)], m_v.at[0], sems[0])
        for j in range(KC):
            cur = j % 2
            nxt = (j + 1) % 2
            if j + 1 < KC:
                descs[nxt] = pltpu.async_copy(
                    m_hbm.at[pl.ds(ebase + (j + 1) * CH, CH)],
                    m_v.at[nxt], sems[nxt])
            descs[cur].wait()
            pltpu.sync_copy(m_v.at[cur], acc_sh.at[idx_v.at[j]], add=True)
        plsc.subcore_barrier()
        pltpu.sync_copy(acc_sh.at[pl.ds(zbase, NZ)],
                        out_hbm.at[pl.ds(c * NT + zbase, NZ)])

    return sc_gather, sc_scatter


def _sc_gather(table, idx2d):
    return _sc_kernels()[0](table, idx2d)


def _sc_scatter(m, idx2d, zeros):
    return _sc_kernels()[1](m, idx2d, zeros)


def _proj_body(x_ref, w_ref, b_ref, o_ref):
    o_ref[...] = jnp.maximum(
        jnp.dot(x_ref[...], w_ref[...], preferred_element_type=jnp.float32)
        + b_ref[...], 0.0)


_proj = pl.pallas_call(
    _proj_body,
    grid=(NT // 2048,),
    in_specs=[pl.BlockSpec((2048, DN), lambda i: (i, 0)),
              pl.BlockSpec((DN, H), lambda i: (0, 0)),
              pl.BlockSpec((1, H), lambda i: (0, 0))],
    out_specs=pl.BlockSpec((2048, H), lambda i: (i, 0)),
    out_shape=jax.ShapeDtypeStruct((NT, H), jnp.float32),
)


def _msg_body(hs4_ref, ea4_ref, w3bd_ref, bmbd_ref, s_ref, o_ref):
    hs4 = hs4_ref[...]
    ea4 = ea4_ref[...]
    acc = jnp.dot(hs4, bmbd_ref[...], preferred_element_type=jnp.float32)
    for d in range(DE):
        scale = jnp.dot(ea4, s_ref[pl.ds(d * H, H), :],
                        preferred_element_type=jnp.float32)
        acc = acc + scale * jnp.dot(
            hs4, w3bd_ref[pl.ds(d * 4 * H, 4 * H), :],
            preferred_element_type=jnp.float32)
    o_ref[...] = acc


_msg = pl.pallas_call(
    _msg_body,
    grid=(ET // 2048,),
    in_specs=[pl.BlockSpec((512, 4 * H), lambda i: (i, 0)),
              pl.BlockSpec((512, 4 * DE), lambda i: (i, 0)),
              pl.BlockSpec((DE * 4 * H, 4 * H), lambda i: (0, 0)),
              pl.BlockSpec((4 * H, 4 * H), lambda i: (0, 0)),
              pl.BlockSpec((DE * H, 4 * H), lambda i: (0, 0))],
    out_specs=pl.BlockSpec((512, 4 * H), lambda i: (i, 0)),
    out_shape=jax.ShapeDtypeStruct((ET // 4, 4 * H), jnp.float32),
)


def _gru_body(p0_ref, p1_ref, h_ref, cb_ref, wr, wz, wn, ur, uz, un,
              br, bz, bi_n, bh_n, o_ref):
    a = jnp.maximum(p0_ref[...] + p1_ref[...] + cb_ref[...], 0.0)
    h = h_ref[...]
    dot = lambda x, w: jnp.dot(x, w[...], preferred_element_type=jnp.float32)
    r = jax.nn.sigmoid(dot(a, wr) + dot(h, ur) + br[...])
    z = jax.nn.sigmoid(dot(a, wz) + dot(h, uz) + bz[...])
    n = jnp.tanh(dot(a, wn) + bi_n[...] + r * (dot(h, un) + bh_n[...]))
    o_ref[...] = (1.0 - z) * n + z * h


_gru = pl.pallas_call(
    _gru_body,
    grid=(NT // 2048,),
    in_specs=[pl.BlockSpec((2048, H), lambda i: (i, 0)),
              pl.BlockSpec((2048, H), lambda i: (i, 0)),
              pl.BlockSpec((2048, H), lambda i: (i, 0)),
              pl.BlockSpec((1, H), lambda i: (0, 0))]
             + [pl.BlockSpec((H, H), lambda i: (0, 0))] * 6
             + [pl.BlockSpec((1, H), lambda i: (0, 0))] * 4,
    out_specs=pl.BlockSpec((2048, H), lambda i: (i, 0)),
    out_shape=jax.ShapeDtypeStruct((NT, H), jnp.float32),
)


CHK = 1024
NCH = N // CHK


def _s2s_body(feat_ref, gidc_ref, wihq, wihr, whh, bb,
              spq, spr, spb, spa, out_ref, mask_ref, ex_ref):
    dot = lambda x, w: jnp.dot(x, w, preferred_element_type=jnp.float32)
    iota_cb = lax.broadcasted_iota(jnp.int32, (CHK, B), 1)
    cdims = (((0,), (0,)), ((), ()))
    bdims = (((1,), (1,)), ((), ()))

    def build(ci, z):
        sl = pl.ds(ci * CHK, CHK)
        mask_ref[sl, :] = (gidc_ref[0, sl, :] == iota_cb).astype(jnp.float32)
        return z

    lax.fori_loop(0, NCH, build, 0)

    q = jnp.zeros((B, 2 * H), jnp.float32)
    rr = jnp.zeros((B, 2 * H), jnp.float32)
    hl = jnp.zeros((B, 2 * H), jnp.float32)
    cl = jnp.zeros((B, 2 * H), jnp.float32)
    for _ in range(3):
        g = (dot(q, wihq[...]) + dot(rr, wihr[...])
             + dot(hl, whh[...]) + bb[...])
        i = jax.nn.sigmoid(g[:, 0:128])
        f = jax.nn.sigmoid(g[:, 128:256])
        gg = jnp.tanh(g[:, 256:384])
        o = jax.nn.sigmoid(g[:, 384:512])
        cl = f * cl + i * gg
        hl = o * jnp.tanh(cl)
        q = hl

        def pass_a(ci, den):
            sl = pl.ds(ci * CHK, CHK)
            mc = mask_ref[sl, :]
            qg = dot(mc, q)
            esc = jnp.sum(feat_ref[0, sl, :] * qg, axis=1, keepdims=True)
            ex = jnp.exp(esc)
            ex_ref[sl, :] = ex
            return den + lax.dot_general(
                ex, mc, cdims, preferred_element_type=jnp.float32)

        den = lax.fori_loop(0, NCH, pass_a, jnp.zeros((1, B), jnp.float32))

        def pass_b(ci, ra):
            sl = pl.ds(ci * CHK, CHK)
            mc = mask_ref[sl, :]
            den_n = lax.dot_general(mc, den, bdims,
                                    preferred_element_type=jnp.float32)
            alpha = ex_ref[sl, :] / den_n
            return ra + lax.dot_general(mc, feat_ref[0, sl, :] * alpha, cdims,
                                        preferred_element_type=jnp.float32)

        rr = lax.fori_loop(0, NCH, pass_b,
                           jnp.zeros((B, 2 * H), jnp.float32))
    out = dot(q, spq[...]) + dot(rr, spr[...]) + spb[...]
    a = spa[0, 0]
    out_ref[0] = jnp.where(out >= 0, out, a * out)


_s2s = pl.pallas_call(
    _s2s_body,
    grid=(G,),
    scratch_shapes=[pltpu.VMEM((N, B), jnp.float32),
                    pltpu.VMEM((N, 1), jnp.float32)],
    in_specs=[pl.BlockSpec((1, N, 2 * H), lambda g: (g, 0, 0)),
              pl.BlockSpec((1, N, 1), lambda g: (g, 0, 0)),
              pl.BlockSpec((2 * H, 512), lambda g: (0, 0)),
              pl.BlockSpec((2 * H, 512), lambda g: (0, 0)),
              pl.BlockSpec((2 * H, 512), lambda g: (0, 0)),
              pl.BlockSpec((1, 512), lambda g: (0, 0)),
              pl.BlockSpec((2 * H, RO), lambda g: (0, 0)),
              pl.BlockSpec((2 * H, RO), lambda g: (0, 0)),
              pl.BlockSpec((1, RO), lambda g: (0, 0)),
              pl.BlockSpec((1, 1), lambda g: (0, 0))],
    out_specs=pl.BlockSpec((1, B, RO), lambda g: (g, 0, 0)),
    out_shape=jax.ShapeDtypeStruct((G, B, RO), jnp.float32),
)


def _mlp_body(xin, eps, ge, pnv,
              w0, b0, a0, w1, b1, a1, w2, b2, a2, w3, b3,
              dw0a, dw0b, dw0c, db0, da0, dw1, db1, da1,
              dw2, db2, da2, dw3, db3,
              y_ref, mu_ref, lv_ref):
    dot = lambda x, w: jnp.dot(x, w[...], preferred_element_type=jnp.float32)
    prelu = lambda x, a: jnp.where(x >= 0, x, a[0, 0] * x)
    x = prelu(dot(xin[...], w0) + b0[...], a0)
    x = prelu(dot(x, w1) + b1[...], a1)
    x = prelu(dot(x, w2) + b2[...], a2)
    x = dot(x, w3) + b3[...]
    mu = jnp.clip(x[:, 0:LAT], -10.0, 10.0)
    lv = jnp.clip(x[:, LAT:2 * LAT], -10.0, 10.0)
    latent = mu + eps[...] * jnp.exp(0.5 * lv)
    y = prelu(dot(latent, dw0a) + dot(ge[...], dw0b)
              + pnv[0, 0] * dw0c[...] + db0[...], da0)
    y = prelu(dot(y, dw1) + db1[...], da1)
    y = prelu(dot(y, dw2) + db2[...], da2)
    y = dot(y, dw3) + db3[...]
    y_ref[...] = jnp.clip(y, -10.0, 10.0)
    mu_ref[...] = mu
    lv_ref[...] = lv


_EI = RO * G + NCLS + 1


def _full(shape):
    return pl.BlockSpec(shape, lambda: tuple(0 for _ in shape))


_mlp = pl.pallas_call(
    _mlp_body,
    in_specs=[_full((B, _EI)), _full((B, LAT)), _full((B, G * RO)),
              _full((1, 1)),
              _full((_EI, PH)), _full((1, PH)), _full((1, 1)),
              _full((PH, PH)), _full((1, PH)), _full((1, 1)),
              _full((PH, PH)), _full((1, PH)), _full((1, 1)),
              _full((PH, 2 * LAT)), _full((1, 2 * LAT)),
              _full((LAT, PH)), _full((G * RO, PH)), _full((1, PH)),
              _full((1, PH)), _full((1, 1)),
              _full((PH, PH)), _full((1, PH)), _full((1, 1)),
              _full((PH, PH)), _full((1, PH)), _full((1, 1)),
              _full((PH, NCLS)), _full((1, NCLS))],
    out_specs=[_full((B, NCLS)), _full((B, LAT)), _full((B, LAT))],
    out_shape=[jax.ShapeDtypeStruct((B, NCLS), jnp.float32),
               jax.ShapeDtypeStruct((B, LAT), jnp.float32),
               jax.ShapeDtypeStruct((B, LAT), jnp.float32)],
)


def kernel(r1_x, r1_e, r1_src, r1_dst, r1_gid, r2_x, r2_e, r2_src, r2_dst,
           r2_gid, pm_x, pm_e, pm_src, pm_dst, pm_gid, labels,
           pos_neg_sample, params):
    p = params
    f32 = jnp.float32
    x_all = jnp.concatenate([r1_x, r2_x, pm_x], axis=0)
    e_all = jnp.concatenate([r1_e, r2_e, pm_e], axis=0)
    src_all = jnp.concatenate(
        [r1_src, r2_src + N, pm_src + 2 * N]).astype(jnp.int32)
    dst_all = jnp.concatenate(
        [r1_dst, r2_dst + N, pm_dst + 2 * N]).astype(jnp.int32)
    src2d = src_all.reshape(NW, KC, CH)
    dst2d = dst_all.reshape(NW, KC, CH)
    zeros_nt = jnp.zeros((NT, H), f32)

    x0 = _proj(x_all, p['proj_W'], p['proj_b'].reshape(1, H))

    eye4 = jnp.eye(4, dtype=f32)
    w3s = p['edge_W'].reshape(DE, H, H)
    w3bd = jnp.concatenate(
        [jnp.kron(eye4, w3s[d]) for d in range(DE)], axis=0)
    bmbd = jnp.kron(eye4, p['edge_b'].reshape(H, H))
    kk = jnp.arange(4 * DE)[None, :, None]
    blk = jnp.arange(4 * H)[None, None, :] // H
    dsel = jnp.arange(DE)[:, None, None]
    smat = (kk == blk * DE + dsel).astype(f32).reshape(DE * 4 * DE, 4 * H)
    wih = p['gru_Wih']
    whh = p['gru_Whh']
    wr, wz, wn = (wih[0:H].T, wih[H:2 * H].T, wih[2 * H:].T)
    ur, uz, un = (whh[0:H].T, whh[H:2 * H].T, whh[2 * H:].T)
    bih = p['gru_bih']
    bhh = p['gru_bhh']
    br = (bih[0:H] + bhh[0:H]).reshape(1, H)
    bz = (bih[H:2 * H] + bhh[H:2 * H]).reshape(1, H)
    bi_n = bih[2 * H:].reshape(1, H)
    bh_n = bhh[2 * H:].reshape(1, H)
    cb = p['conv_b'].reshape(1, H)

    h = x0
    for _ in range(3):
        hsrc = _sc_gather(h, src2d)
        m4 = _msg(hsrc.reshape(ET // 4, 4 * H), e_all.reshape(ET // 4, 4 * DE),
                  w3bd, bmbd, smat)
        parts = _sc_scatter(m4.reshape(ET, H), dst2d, zeros_nt)
        h = _gru(parts[0:NT], parts[NT:2 * NT], h, cb, wr, wz, wn,
                 ur, uz, un, br, bz, bi_n, bh_n)

    gid_all = jnp.stack([r1_gid, r2_gid, pm_gid]).astype(jnp.int32)
    wihT = p['lstm_Wih'].T
    feat = jnp.concatenate(
        [x0.reshape(G, N, H), h.reshape(G, N, H)], axis=2)
    go = _s2s(feat, gid_all.reshape(G, N, 1),
              wihT[0:2 * H], wihT[2 * H:4 * H],
              p['lstm_Whh'].T,
              (p['lstm_bih'] + p['lstm_bhh']).reshape(1, 8 * H),
              p['sp_W'][0:2 * H], p['sp_W'][2 * H:4 * H],
              p['sp_b'].reshape(1, RO), p['sp_a'].reshape(1, 1))

    ge = jnp.transpose(go, (1, 0, 2)).reshape(B, G * RO)
    pos = jnp.asarray(pos_neg_sample).astype(f32)
    pn = jnp.zeros((B, 1), f32) + pos
    xin = jnp.concatenate([labels, ge, pn], axis=1)
    eps = jax.random.normal(jax.random.key(42), (B, LAT), dtype=f32)
    dw0 = p['dec_W0']
    y, mu, lv = _mlp(
        xin, eps, ge, pos.reshape(1, 1),
        p['enc_W0'], p['enc_b0'].reshape(1, PH), p['enc_a0'].reshape(1, 1),
        p['enc_W1'], p['enc_b1'].reshape(1, PH), p['enc_a1'].reshape(1, 1),
        p['enc_W2'], p['enc_b2'].reshape(1, PH), p['enc_a2'].reshape(1, 1),
        p['enc_W3'], p['enc_b3'].reshape(1, 2 * LAT),
        dw0[0:LAT], dw0[LAT:LAT + G * RO], dw0[LAT + G * RO:],
        p['dec_b0'].reshape(1, PH), p['dec_a0'].reshape(1, 1),
        p['dec_W1'], p['dec_b1'].reshape(1, PH), p['dec_a1'].reshape(1, 1),
        p['dec_W2'], p['dec_b2'].reshape(1, PH), p['dec_a2'].reshape(1, 1),
        p['dec_W3'], p['dec_b3'].reshape(1, NCLS))
    return (y, mu, lv)


# speedup vs baseline: 1.2578x; 1.0289x over previous
import functools

import jax
import jax.numpy as jnp
from jax import lax
from jax.experimental import pallas as pl
from jax.experimental.pallas import tpu as pltpu
from jax.experimental.pallas import tpu_sc as plsc

H = 64
DN = 128
DE = 16
RO = 1024
PH = 512
LAT = 128
NCLS = 100
B = 256
N = 8192
E = 16384
G = 3
NT = G * N
ET = G * E

SC_CORES = 2
SC_SUB = 16
NW = SC_CORES * SC_SUB
CH = 128
RW = ET // NW
KC = RW // CH
NZ = NT // SC_SUB

@functools.cache
def _sc_kernels():
    mesh = plsc.VectorSubcoreMesh(
        core_axis_name="c", subcore_axis_name="s",
        num_cores=SC_CORES, num_subcores=SC_SUB)

    @functools.partial(
        pl.kernel,
        out_type=jax.ShapeDtypeStruct((ET, H), jnp.float32),
        mesh=mesh,
        compiler_params=pltpu.CompilerParams(use_tc_tiling_on_sc=False),
        scratch_types=[
            pltpu.VMEM((KC, CH), jnp.int32),
            pltpu.VMEM((RW, H), jnp.float32),
            pltpu.SemaphoreType.DMA,
        ],
    )
    def sc_gather(table_hbm, idx_hbm, out_hbm, idx_v, rows_v, sem):
        c = lax.axis_index("c")
        s = lax.axis_index("s")
        wid = s * SC_CORES + c
        pltpu.sync_copy(idx_hbm.at[wid], idx_v)
        descs = []
        for j in range(KC):
            descs.append(pltpu.async_copy(
                table_hbm.at[idx_v.at[j]], rows_v.at[pl.ds(j * CH, CH)], sem))
        for d in descs:
            d.wait()
        pltpu.sync_copy(rows_v, out_hbm.at[pl.ds(wid * RW, RW)])

    @functools.partial(
        pl.kernel,
        out_type=jax.ShapeDtypeStruct((SC_CORES * NT, H), jnp.float32),
        mesh=mesh,
        compiler_params=pltpu.CompilerParams(use_tc_tiling_on_sc=False),
        scratch_types=[
            pltpu.VMEM((KC, CH), jnp.int32),
            pltpu.VMEM((2, CH, H), jnp.float32),
            pltpu.VMEM_SHARED((NT, H), jnp.float32),
            pltpu.SemaphoreType.DMA,
            pltpu.SemaphoreType.DMA,
        ],
    )
    def sc_scatter(m_hbm, idx_hbm, zeros_hbm, out_hbm, idx_v, m_v, acc_sh,
                   sem0, sem1):
        c = lax.axis_index("c")
        s = lax.axis_index("s")
        zbase = s * NZ
        pltpu.sync_copy(zeros_hbm.at[pl.ds(zbase, NZ)],
                        acc_sh.at[pl.ds(zbase, NZ)])
        ebase = c * (ET // SC_CORES) + s * RW
        pltpu.sync_copy(idx_hbm.at[c * SC_SUB + s], idx_v)
        plsc.subcore_barrier()
        sems = (sem0, sem1)
        descs = [None, None]
        descs[0] = pltpu.async_copy(
            m_hbm.at[pl.ds(ebase, CH)], m_v.at[0], sems[0])
        for j in range(KC):
            cur = j % 2
            nxt = (j + 1) % 2
            if j + 1 < KC:
                descs[nxt] = pltpu.async_copy(
                    m_hbm.at[pl.ds(ebase + (j + 1) * CH, CH)],
                    m_v.at[nxt], sems[nxt])
            descs[cur].wait()
            pltpu.sync_copy(m_v.at[cur], acc_sh.at[idx_v.at[j]], add=True)
        plsc.subcore_barrier()
        pltpu.sync_copy(acc_sh.at[pl.ds(zbase, NZ)],
                        out_hbm.at[pl.ds(c * NT + zbase, NZ)])

    return sc_gather, sc_scatter


def _sc_gather(table, idx2d):
    return _sc_kernels()[0](table, idx2d)


def _sc_scatter(m, idx2d, zeros):
    return _sc_kernels()[1](m, idx2d, zeros)


def _proj_body(x_ref, w_ref, b_ref, o_ref):
    o_ref[...] = jnp.maximum(
        jnp.dot(x_ref[...], w_ref[...], preferred_element_type=jnp.float32)
        + b_ref[...], 0.0)


_proj = pl.pallas_call(
    _proj_body,
    grid=(NT // 2048,),
    in_specs=[pl.BlockSpec((2048, DN), lambda i: (i, 0)),
              pl.BlockSpec((DN, H), lambda i: (0, 0)),
              pl.BlockSpec((1, H), lambda i: (0, 0))],
    out_specs=pl.BlockSpec((2048, H), lambda i: (i, 0)),
    out_shape=jax.ShapeDtypeStruct((NT, H), jnp.float32),
)


def _msg_body(hs4_ref, ea4_ref, w3bd_ref, bmbd_ref, s_ref, o_ref):
    hs4 = hs4_ref[...]
    ea4 = ea4_ref[...]
    acc = jnp.dot(hs4, bmbd_ref[...], preferred_element_type=jnp.float32)
    for d in range(DE):
        scale = jnp.dot(ea4, s_ref[pl.ds(d * H, H), :],
                        preferred_element_type=jnp.float32)
        acc = acc + scale * jnp.dot(
            hs4, w3bd_ref[pl.ds(d * 4 * H, 4 * H), :],
            preferred_element_type=jnp.float32)
    o_ref[...] = acc


_msg = pl.pallas_call(
    _msg_body,
    grid=(ET // 2048,),
    in_specs=[pl.BlockSpec((512, 4 * H), lambda i: (i, 0)),
              pl.BlockSpec((512, 4 * DE), lambda i: (i, 0)),
              pl.BlockSpec((DE * 4 * H, 4 * H), lambda i: (0, 0)),
              pl.BlockSpec((4 * H, 4 * H), lambda i: (0, 0)),
              pl.BlockSpec((DE * H, 4 * H), lambda i: (0, 0))],
    out_specs=pl.BlockSpec((512, 4 * H), lambda i: (i, 0)),
    out_shape=jax.ShapeDtypeStruct((ET // 4, 4 * H), jnp.float32),
)


def _gru_body(p0_ref, p1_ref, h_ref, cb_ref, wr, wz, wn, ur, uz, un,
              br, bz, bi_n, bh_n, o_ref):
    a = jnp.maximum(p0_ref[...] + p1_ref[...] + cb_ref[...], 0.0)
    h = h_ref[...]
    dot = lambda x, w: jnp.dot(x, w[...], preferred_element_type=jnp.float32)
    r = jax.nn.sigmoid(dot(a, wr) + dot(h, ur) + br[...])
    z = jax.nn.sigmoid(dot(a, wz) + dot(h, uz) + bz[...])
    n = jnp.tanh(dot(a, wn) + bi_n[...] + r * (dot(h, un) + bh_n[...]))
    o_ref[...] = (1.0 - z) * n + z * h


_gru = pl.pallas_call(
    _gru_body,
    grid=(NT // 2048,),
    in_specs=[pl.BlockSpec((2048, H), lambda i: (i, 0)),
              pl.BlockSpec((2048, H), lambda i: (i, 0)),
              pl.BlockSpec((2048, H), lambda i: (i, 0)),
              pl.BlockSpec((1, H), lambda i: (0, 0))]
             + [pl.BlockSpec((H, H), lambda i: (0, 0))] * 6
             + [pl.BlockSpec((1, H), lambda i: (0, 0))] * 4,
    out_specs=pl.BlockSpec((2048, H), lambda i: (i, 0)),
    out_shape=jax.ShapeDtypeStruct((NT, H), jnp.float32),
)


CHK = 2048
NCH = N // CHK


def _s2s_body(feat_ref, gidc_ref, wihq, wihr, whh, bb,
              spq, spr, spb, spa, out_ref, mask_ref, ex_ref):
    dot = lambda x, w: jnp.dot(x, w, preferred_element_type=jnp.float32)
    iota_cb = lax.broadcasted_iota(jnp.int32, (CHK, B), 1)
    cdims = (((0,), (0,)), ((), ()))
    bdims = (((1,), (1,)), ((), ()))

    def build(ci, z):
        sl = pl.ds(ci * CHK, CHK)
        mask_ref[sl, :] = (gidc_ref[0, sl, :] == iota_cb).astype(jnp.float32)
        return z

    lax.fori_loop(0, NCH, build, 0)

    q = jnp.zeros((B, 2 * H), jnp.float32)
    rr = jnp.zeros((B, 2 * H), jnp.float32)
    hl = jnp.zeros((B, 2 * H), jnp.float32)
    cl = jnp.zeros((B, 2 * H), jnp.float32)
    for _ in range(3):
        g = (dot(q, wihq[...]) + dot(rr, wihr[...])
             + dot(hl, whh[...]) + bb[...])
        i = jax.nn.sigmoid(g[:, 0:128])
        f = jax.nn.sigmoid(g[:, 128:256])
        gg = jnp.tanh(g[:, 256:384])
        o = jax.nn.sigmoid(g[:, 384:512])
        cl = f * cl + i * gg
        hl = o * jnp.tanh(cl)
        q = hl

        def pass_a(ci, den):
            sl = pl.ds(ci * CHK, CHK)
            mc = mask_ref[sl, :]
            qg = dot(mc, q)
            esc = jnp.sum(feat_ref[0, sl, :] * qg, axis=1, keepdims=True)
            ex = jnp.exp(esc)
            ex_ref[sl, :] = ex
            return den + lax.dot_general(
                ex, mc, cdims, preferred_element_type=jnp.float32)

        den = lax.fori_loop(0, NCH, pass_a, jnp.zeros((1, B), jnp.float32))

        def pass_b(ci, ra):
            sl = pl.ds(ci * CHK, CHK)
            mc = mask_ref[sl, :]
            den_n = lax.dot_general(mc, den, bdims,
                                    preferred_element_type=jnp.float32)
            alpha = ex_ref[sl, :] / den_n
            return ra + lax.dot_general(mc, feat_ref[0, sl, :] * alpha, cdims,
                                        preferred_element_type=jnp.float32)

        rr = lax.fori_loop(0, NCH, pass_b,
                           jnp.zeros((B, 2 * H), jnp.float32))
    out = dot(q, spq[...]) + dot(rr, spr[...]) + spb[...]
    a = spa[0, 0]
    out_ref[0] = jnp.where(out >= 0, out, a * out)


_s2s = pl.pallas_call(
    _s2s_body,
    grid=(G,),
    scratch_shapes=[pltpu.VMEM((N, B), jnp.float32),
                    pltpu.VMEM((N, 1), jnp.float32)],
    in_specs=[pl.BlockSpec((1, N, 2 * H), lambda g: (g, 0, 0)),
              pl.BlockSpec((1, N, 1), lambda g: (g, 0, 0)),
              pl.BlockSpec((2 * H, 512), lambda g: (0, 0)),
              pl.BlockSpec((2 * H, 512), lambda g: (0, 0)),
              pl.BlockSpec((2 * H, 512), lambda g: (0, 0)),
              pl.BlockSpec((1, 512), lambda g: (0, 0)),
              pl.BlockSpec((2 * H, RO), lambda g: (0, 0)),
              pl.BlockSpec((2 * H, RO), lambda g: (0, 0)),
              pl.BlockSpec((1, RO), lambda g: (0, 0)),
              pl.BlockSpec((1, 1), lambda g: (0, 0))],
    out_specs=pl.BlockSpec((1, B, RO), lambda g: (g, 0, 0)),
    out_shape=jax.ShapeDtypeStruct((G, B, RO), jnp.float32),
)


def _mlp_body(xin, eps, ge, pnv,
              w0, b0, a0, w1, b1, a1, w2, b2, a2, w3, b3,
              dw0a, dw0b, dw0c, db0, da0, dw1, db1, da1,
              dw2, db2, da2, dw3, db3,
              y_ref, mu_ref, lv_ref):
    dot = lambda x, w: jnp.dot(x, w[...], preferred_element_type=jnp.float32)
    prelu = lambda x, a: jnp.where(x >= 0, x, a[0, 0] * x)
    x = prelu(dot(xin[...], w0) + b0[...], a0)
    x = prelu(dot(x, w1) + b1[...], a1)
    x = prelu(dot(x, w2) + b2[...], a2)
    x = dot(x, w3) + b3[...]
    mu = jnp.clip(x[:, 0:LAT], -10.0, 10.0)
    lv = jnp.clip(x[:, LAT:2 * LAT], -10.0, 10.0)
    latent = mu + eps[...] * jnp.exp(0.5 * lv)
    y = prelu(dot(latent, dw0a) + dot(ge[...], dw0b)
              + pnv[0, 0] * dw0c[...] + db0[...], da0)
    y = prelu(dot(y, dw1) + db1[...], da1)
    y = prelu(dot(y, dw2) + db2[...], da2)
    y = dot(y, dw3) + db3[...]
    y_ref[...] = jnp.clip(y, -10.0, 10.0)
    mu_ref[...] = mu
    lv_ref[...] = lv


_EI = RO * G + NCLS + 1


def _full(shape):
    return pl.BlockSpec(shape, lambda: tuple(0 for _ in shape))


_mlp = pl.pallas_call(
    _mlp_body,
    in_specs=[_full((B, _EI)), _full((B, LAT)), _full((B, G * RO)),
              _full((1, 1)),
              _full((_EI, PH)), _full((1, PH)), _full((1, 1)),
              _full((PH, PH)), _full((1, PH)), _full((1, 1)),
              _full((PH, PH)), _full((1, PH)), _full((1, 1)),
              _full((PH, 2 * LAT)), _full((1, 2 * LAT)),
              _full((LAT, PH)), _full((G * RO, PH)), _full((1, PH)),
              _full((1, PH)), _full((1, 1)),
              _full((PH, PH)), _full((1, PH)), _full((1, 1)),
              _full((PH, PH)), _full((1, PH)), _full((1, 1)),
              _full((PH, NCLS)), _full((1, NCLS))],
    out_specs=[_full((B, NCLS)), _full((B, LAT)), _full((B, LAT))],
    out_shape=[jax.ShapeDtypeStruct((B, NCLS), jnp.float32),
               jax.ShapeDtypeStruct((B, LAT), jnp.float32),
               jax.ShapeDtypeStruct((B, LAT), jnp.float32)],
)


def kernel(r1_x, r1_e, r1_src, r1_dst, r1_gid, r2_x, r2_e, r2_src, r2_dst,
           r2_gid, pm_x, pm_e, pm_src, pm_dst, pm_gid, labels,
           pos_neg_sample, params):
    p = params
    f32 = jnp.float32
    x_all = jnp.concatenate([r1_x, r2_x, pm_x], axis=0)
    e_all = jnp.concatenate([r1_e, r2_e, pm_e], axis=0)
    src_all = jnp.concatenate(
        [r1_src, r2_src + N, pm_src + 2 * N]).astype(jnp.int32)
    dst_all = jnp.concatenate(
        [r1_dst, r2_dst + N, pm_dst + 2 * N]).astype(jnp.int32)
    src2d = src_all.reshape(NW, KC, CH)
    dst2d = dst_all.reshape(NW, KC, CH)
    zeros_nt = jnp.zeros((NT, H), f32)

    x0 = _proj(x_all, p['proj_W'], p['proj_b'].reshape(1, H))

    eye4 = jnp.eye(4, dtype=f32)
    w3s = p['edge_W'].reshape(DE, H, H)
    w3bd = jnp.concatenate(
        [jnp.kron(eye4, w3s[d]) for d in range(DE)], axis=0)
    bmbd = jnp.kron(eye4, p['edge_b'].reshape(H, H))
    kk = jnp.arange(4 * DE)[None, :, None]
    blk = jnp.arange(4 * H)[None, None, :] // H
    dsel = jnp.arange(DE)[:, None, None]
    smat = (kk == blk * DE + dsel).astype(f32).reshape(DE * 4 * DE, 4 * H)
    wih = p['gru_Wih']
    whh = p['gru_Whh']
    wr, wz, wn = (wih[0:H].T, wih[H:2 * H].T, wih[2 * H:].T)
    ur, uz, un = (whh[0:H].T, whh[H:2 * H].T, whh[2 * H:].T)
    bih = p['gru_bih']
    bhh = p['gru_bhh']
    br = (bih[0:H] + bhh[0:H]).reshape(1, H)
    bz = (bih[H:2 * H] + bhh[H:2 * H]).reshape(1, H)
    bi_n = bih[2 * H:].reshape(1, H)
    bh_n = bhh[2 * H:].reshape(1, H)
    cb = p['conv_b'].reshape(1, H)

    h = x0
    for _ in range(3):
        hsrc = _sc_gather(h, src2d)
        m4 = _msg(hsrc.reshape(ET // 4, 4 * H), e_all.reshape(ET // 4, 4 * DE),
                  w3bd, bmbd, smat)
        parts = _sc_scatter(m4.reshape(ET, H), dst2d, zeros_nt)
        h = _gru(parts[0:NT], parts[NT:2 * NT], h, cb, wr, wz, wn,
                 ur, uz, un, br, bz, bi_n, bh_n)

    gid_all = jnp.stack([r1_gid, r2_gid, pm_gid]).astype(jnp.int32)
    wihT = p['lstm_Wih'].T
    feat = jnp.concatenate(
        [x0.reshape(G, N, H), h.reshape(G, N, H)], axis=2)
    go = _s2s(feat, gid_all.reshape(G, N, 1),
              wihT[0:2 * H], wihT[2 * H:4 * H],
              p['lstm_Whh'].T,
              (p['lstm_bih'] + p['lstm_bhh']).reshape(1, 8 * H),
              p['sp_W'][0:2 * H], p['sp_W'][2 * H:4 * H],
              p['sp_b'].reshape(1, RO), p['sp_a'].reshape(1, 1))

    ge = jnp.transpose(go, (1, 0, 2)).reshape(B, G * RO)
    pos = jnp.asarray(pos_neg_sample).astype(f32)
    pn = jnp.zeros((B, 1), f32) + pos
    xin = jnp.concatenate([labels, ge, pn], axis=1)
    eps = jax.random.normal(jax.random.key(42), (B, LAT), dtype=f32)
    dw0 = p['dec_W0']
    y, mu, lv = _mlp(
        xin, eps, ge, pos.reshape(1, 1),
        p['enc_W0'], p['enc_b0'].reshape(1, PH), p['enc_a0'].reshape(1, 1),
        p['enc_W1'], p['enc_b1'].reshape(1, PH), p['enc_a1'].reshape(1, 1),
        p['enc_W2'], p['enc_b2'].reshape(1, PH), p['enc_a2'].reshape(1, 1),
        p['enc_W3'], p['enc_b3'].reshape(1, 2 * LAT),
        dw0[0:LAT], dw0[LAT:LAT + G * RO], dw0[LAT + G * RO:],
        p['dec_b0'].reshape(1, PH), p['dec_a0'].reshape(1, 1),
        p['dec_W1'], p['dec_b1'].reshape(1, PH), p['dec_a1'].reshape(1, 1),
        p['dec_W2'], p['dec_b2'].reshape(1, PH), p['dec_a2'].reshape(1, 1),
        p['dec_W3'], p['dec_b3'].reshape(1, NCLS))
    return (y, mu, lv)


# speedup vs baseline: 1.2741x; 1.0129x over previous
import functools

import jax
import jax.numpy as jnp
from jax import lax
from jax.experimental import pallas as pl
from jax.experimental.pallas import tpu as pltpu
from jax.experimental.pallas import tpu_sc as plsc

H = 64
DN = 128
DE = 16
RO = 1024
PH = 512
LAT = 128
NCLS = 100
B = 256
N = 8192
E = 16384
G = 3
NT = G * N
ET = G * E

SC_CORES = 2
SC_SUB = 16
NW = SC_CORES * SC_SUB
CH = 128
RW = ET // NW
KC = RW // CH
NZ = NT // SC_SUB

@functools.cache
def _sc_kernels():
    mesh = plsc.VectorSubcoreMesh(
        core_axis_name="c", subcore_axis_name="s",
        num_cores=SC_CORES, num_subcores=SC_SUB)

    @functools.partial(
        pl.kernel,
        out_type=jax.ShapeDtypeStruct((ET, H), jnp.float32),
        mesh=mesh,
        compiler_params=pltpu.CompilerParams(use_tc_tiling_on_sc=False),
        scratch_types=[
            pltpu.VMEM((KC, CH), jnp.int32),
            pltpu.VMEM((RW, H), jnp.float32),
            pltpu.SemaphoreType.DMA,
        ],
    )
    def sc_gather(table_hbm, idx_hbm, out_hbm, idx_v, rows_v, sem):
        c = lax.axis_index("c")
        s = lax.axis_index("s")
        wid = s * SC_CORES + c
        pltpu.sync_copy(idx_hbm.at[wid], idx_v)
        descs = []
        for j in range(KC):
            descs.append(pltpu.async_copy(
                table_hbm.at[idx_v.at[j]], rows_v.at[pl.ds(j * CH, CH)], sem))
        for d in descs:
            d.wait()
        pltpu.sync_copy(rows_v, out_hbm.at[pl.ds(wid * RW, RW)])

    @functools.partial(
        pl.kernel,
        out_type=jax.ShapeDtypeStruct((SC_CORES * NT, H), jnp.float32),
        mesh=mesh,
        compiler_params=pltpu.CompilerParams(use_tc_tiling_on_sc=False),
        scratch_types=[
            pltpu.VMEM((KC, CH), jnp.int32),
            pltpu.VMEM((2, CH, H), jnp.float32),
            pltpu.VMEM_SHARED((NT, H), jnp.float32),
            pltpu.SemaphoreType.DMA,
            pltpu.SemaphoreType.DMA,
        ],
    )
    def sc_scatter(m_hbm, idx_hbm, zeros_hbm, out_hbm, idx_v, m_v, acc_sh,
                   sem0, sem1):
        c = lax.axis_index("c")
        s = lax.axis_index("s")
        zbase = s * NZ
        pltpu.sync_copy(zeros_hbm.at[pl.ds(zbase, NZ)],
                        acc_sh.at[pl.ds(zbase, NZ)])
        ebase = c * (ET // SC_CORES) + s * RW
        pltpu.sync_copy(idx_hbm.at[c * SC_SUB + s], idx_v)
        plsc.subcore_barrier()
        sems = (sem0, sem1)
        descs = [None, None]
        descs[0] = pltpu.async_copy(
            m_hbm.at[pl.ds(ebase, CH)], m_v.at[0], sems[0])
        for j in range(KC):
            cur = j % 2
            nxt = (j + 1) % 2
            if j + 1 < KC:
                descs[nxt] = pltpu.async_copy(
                    m_hbm.at[pl.ds(ebase + (j + 1) * CH, CH)],
                    m_v.at[nxt], sems[nxt])
            descs[cur].wait()
            pltpu.sync_copy(m_v.at[cur], acc_sh.at[idx_v.at[j]], add=True)
        plsc.subcore_barrier()
        pltpu.sync_copy(acc_sh.at[pl.ds(zbase, NZ)],
                        out_hbm.at[pl.ds(c * NT + zbase, NZ)])

    return sc_gather, sc_scatter


def _sc_gather(table, idx2d):
    return _sc_kernels()[0](table, idx2d)


def _sc_scatter(m, idx2d, zeros):
    return _sc_kernels()[1](m, idx2d, zeros)


def _proj_body(x_ref, w_ref, b_ref, o_ref):
    o_ref[...] = jnp.maximum(
        jnp.dot(x_ref[...], w_ref[...], preferred_element_type=jnp.float32)
        + b_ref[...], 0.0)


_proj = pl.pallas_call(
    _proj_body,
    grid=(NT // 2048,),
    in_specs=[pl.BlockSpec((2048, DN), lambda i: (i, 0)),
              pl.BlockSpec((DN, H), lambda i: (0, 0)),
              pl.BlockSpec((1, H), lambda i: (0, 0))],
    out_specs=pl.BlockSpec((2048, H), lambda i: (i, 0)),
    out_shape=jax.ShapeDtypeStruct((NT, H), jnp.float32),
)


def _msg_body(hs4_ref, ea4_ref, w3bd_ref, bmbd_ref, s_ref, o_ref):
    hs4 = hs4_ref[...]
    ea4 = ea4_ref[...]
    acc = jnp.dot(hs4, bmbd_ref[...], preferred_element_type=jnp.float32)
    for d in range(DE):
        scale = jnp.dot(ea4, s_ref[pl.ds(d * H, H), :],
                        preferred_element_type=jnp.float32)
        acc = acc + scale * jnp.dot(
            hs4, w3bd_ref[pl.ds(d * 4 * H, 4 * H), :],
            preferred_element_type=jnp.float32)
    o_ref[...] = acc


_msg = pl.pallas_call(
    _msg_body,
    grid=(ET // 2048,),
    in_specs=[pl.BlockSpec((512, 4 * H), lambda i: (i, 0)),
              pl.BlockSpec((512, 4 * DE), lambda i: (i, 0)),
              pl.BlockSpec((DE * 4 * H, 4 * H), lambda i: (0, 0)),
              pl.BlockSpec((4 * H, 4 * H), lambda i: (0, 0)),
              pl.BlockSpec((DE * H, 4 * H), lambda i: (0, 0))],
    out_specs=pl.BlockSpec((512, 4 * H), lambda i: (i, 0)),
    out_shape=jax.ShapeDtypeStruct((ET // 4, 4 * H), jnp.float32),
)


def _gru_body(p0_ref, p1_ref, h_ref, cb_ref, wr, wz, wn, ur, uz, un,
              br, bz, bi_n, bh_n, o_ref):
    a = jnp.maximum(p0_ref[...] + p1_ref[...] + cb_ref[...], 0.0)
    h = h_ref[...]
    dot = lambda x, w: jnp.dot(x, w[...], preferred_element_type=jnp.float32)
    r = jax.nn.sigmoid(dot(a, wr) + dot(h, ur) + br[...])
    z = jax.nn.sigmoid(dot(a, wz) + dot(h, uz) + bz[...])
    n = jnp.tanh(dot(a, wn) + bi_n[...] + r * (dot(h, un) + bh_n[...]))
    o_ref[...] = (1.0 - z) * n + z * h


_gru = pl.pallas_call(
    _gru_body,
    grid=(NT // 2048,),
    in_specs=[pl.BlockSpec((2048, H), lambda i: (i, 0)),
              pl.BlockSpec((2048, H), lambda i: (i, 0)),
              pl.BlockSpec((2048, H), lambda i: (i, 0)),
              pl.BlockSpec((1, H), lambda i: (0, 0))]
             + [pl.BlockSpec((H, H), lambda i: (0, 0))] * 6
             + [pl.BlockSpec((1, H), lambda i: (0, 0))] * 4,
    out_specs=pl.BlockSpec((2048, H), lambda i: (i, 0)),
    out_shape=jax.ShapeDtypeStruct((NT, H), jnp.float32),
)


CHK = 4096
NCH = N // CHK


def _s2s_body(feat_ref, gidc_ref, wihq, wihr, whh, bb,
              spq, spr, spb, spa, out_ref, mask_ref, ex_ref):
    dot = lambda x, w: jnp.dot(x, w, preferred_element_type=jnp.float32)
    iota_cb = lax.broadcasted_iota(jnp.int32, (CHK, B), 1)
    cdims = (((0,), (0,)), ((), ()))
    bdims = (((1,), (1,)), ((), ()))

    def build(ci, z):
        sl = pl.ds(ci * CHK, CHK)
        mask_ref[sl, :] = (gidc_ref[0, sl, :] == iota_cb).astype(jnp.float32)
        return z

    lax.fori_loop(0, NCH, build, 0)

    q = jnp.zeros((B, 2 * H), jnp.float32)
    rr = jnp.zeros((B, 2 * H), jnp.float32)
    hl = jnp.zeros((B, 2 * H), jnp.float32)
    cl = jnp.zeros((B, 2 * H), jnp.float32)
    for _ in range(3):
        g = (dot(q, wihq[...]) + dot(rr, wihr[...])
             + dot(hl, whh[...]) + bb[...])
        i = jax.nn.sigmoid(g[:, 0:128])
        f = jax.nn.sigmoid(g[:, 128:256])
        gg = jnp.tanh(g[:, 256:384])
        o = jax.nn.sigmoid(g[:, 384:512])
        cl = f * cl + i * gg
        hl = o * jnp.tanh(cl)
        q = hl

        def pass_a(ci, den):
            sl = pl.ds(ci * CHK, CHK)
            mc = mask_ref[sl, :]
            qg = dot(mc, q)
            esc = jnp.sum(feat_ref[0, sl, :] * qg, axis=1, keepdims=True)
            ex = jnp.exp(esc)
            ex_ref[sl, :] = ex
            return den + lax.dot_general(
                ex, mc, cdims, preferred_element_type=jnp.float32)

        den = lax.fori_loop(0, NCH, pass_a, jnp.zeros((1, B), jnp.float32))

        def pass_b(ci, ra):
            sl = pl.ds(ci * CHK, CHK)
            mc = mask_ref[sl, :]
            den_n = lax.dot_general(mc, den, bdims,
                                    preferred_element_type=jnp.float32)
            alpha = ex_ref[sl, :] / den_n
            return ra + lax.dot_general(mc, feat_ref[0, sl, :] * alpha, cdims,
                                        preferred_element_type=jnp.float32)

        rr = lax.fori_loop(0, NCH, pass_b,
                           jnp.zeros((B, 2 * H), jnp.float32))
    out = dot(q, spq[...]) + dot(rr, spr[...]) + spb[...]
    a = spa[0, 0]
    out_ref[0] = jnp.where(out >= 0, out, a * out)


_s2s = pl.pallas_call(
    _s2s_body,
    grid=(G,),
    scratch_shapes=[pltpu.VMEM((N, B), jnp.float32),
                    pltpu.VMEM((N, 1), jnp.float32)],
    in_specs=[pl.BlockSpec((1, N, 2 * H), lambda g: (g, 0, 0)),
              pl.BlockSpec((1, N, 1), lambda g: (g, 0, 0)),
              pl.BlockSpec((2 * H, 512), lambda g: (0, 0)),
              pl.BlockSpec((2 * H, 512), lambda g: (0, 0)),
              pl.BlockSpec((2 * H, 512), lambda g: (0, 0)),
              pl.BlockSpec((1, 512), lambda g: (0, 0)),
              pl.BlockSpec((2 * H, RO), lambda g: (0, 0)),
              pl.BlockSpec((2 * H, RO), lambda g: (0, 0)),
              pl.BlockSpec((1, RO), lambda g: (0, 0)),
              pl.BlockSpec((1, 1), lambda g: (0, 0))],
    out_specs=pl.BlockSpec((1, B, RO), lambda g: (g, 0, 0)),
    out_shape=jax.ShapeDtypeStruct((G, B, RO), jnp.float32),
)


def _mlp_body(xin, eps, ge, pnv,
              w0, b0, a0, w1, b1, a1, w2, b2, a2, w3, b3,
              dw0a, dw0b, dw0c, db0, da0, dw1, db1, da1,
              dw2, db2, da2, dw3, db3,
              y_ref, mu_ref, lv_ref):
    dot = lambda x, w: jnp.dot(x, w[...], preferred_element_type=jnp.float32)
    prelu = lambda x, a: jnp.where(x >= 0, x, a[0, 0] * x)
    x = prelu(dot(xin[...], w0) + b0[...], a0)
    x = prelu(dot(x, w1) + b1[...], a1)
    x = prelu(dot(x, w2) + b2[...], a2)
    x = dot(x, w3) + b3[...]
    mu = jnp.clip(x[:, 0:LAT], -10.0, 10.0)
    lv = jnp.clip(x[:, LAT:2 * LAT], -10.0, 10.0)
    latent = mu + eps[...] * jnp.exp(0.5 * lv)
    y = prelu(dot(latent, dw0a) + dot(ge[...], dw0b)
              + pnv[0, 0] * dw0c[...] + db0[...], da0)
    y = prelu(dot(y, dw1) + db1[...], da1)
    y = prelu(dot(y, dw2) + db2[...], da2)
    y = dot(y, dw3) + db3[...]
    y_ref[...] = jnp.clip(y, -10.0, 10.0)
    mu_ref[...] = mu
    lv_ref[...] = lv


_EI = RO * G + NCLS + 1


def _full(shape):
    return pl.BlockSpec(shape, lambda: tuple(0 for _ in shape))


_mlp = pl.pallas_call(
    _mlp_body,
    in_specs=[_full((B, _EI)), _full((B, LAT)), _full((B, G * RO)),
              _full((1, 1)),
              _full((_EI, PH)), _full((1, PH)), _full((1, 1)),
              _full((PH, PH)), _full((1, PH)), _full((1, 1)),
              _full((PH, PH)), _full((1, PH)), _full((1, 1)),
              _full((PH, 2 * LAT)), _full((1, 2 * LAT)),
              _full((LAT, PH)), _full((G * RO, PH)), _full((1, PH)),
              _full((1, PH)), _full((1, 1)),
              _full((PH, PH)), _full((1, PH)), _full((1, 1)),
              _full((PH, PH)), _full((1, PH)), _full((1, 1)),
              _full((PH, NCLS)), _full((1, NCLS))],
    out_specs=[_full((B, NCLS)), _full((B, LAT)), _full((B, LAT))],
    out_shape=[jax.ShapeDtypeStruct((B, NCLS), jnp.float32),
               jax.ShapeDtypeStruct((B, LAT), jnp.float32),
               jax.ShapeDtypeStruct((B, LAT), jnp.float32)],
)


def kernel(r1_x, r1_e, r1_src, r1_dst, r1_gid, r2_x, r2_e, r2_src, r2_dst,
           r2_gid, pm_x, pm_e, pm_src, pm_dst, pm_gid, labels,
           pos_neg_sample, params):
    p = params
    f32 = jnp.float32
    x_all = jnp.concatenate([r1_x, r2_x, pm_x], axis=0)
    e_all = jnp.concatenate([r1_e, r2_e, pm_e], axis=0)
    src_all = jnp.concatenate(
        [r1_src, r2_src + N, pm_src + 2 * N]).astype(jnp.int32)
    dst_all = jnp.concatenate(
        [r1_dst, r2_dst + N, pm_dst + 2 * N]).astype(jnp.int32)
    src2d = src_all.reshape(NW, KC, CH)
    dst2d = dst_all.reshape(NW, KC, CH)
    zeros_nt = jnp.zeros((NT, H), f32)

    x0 = _proj(x_all, p['proj_W'], p['proj_b'].reshape(1, H))

    eye4 = jnp.eye(4, dtype=f32)
    w3s = p['edge_W'].reshape(DE, H, H)
    w3bd = jnp.concatenate(
        [jnp.kron(eye4, w3s[d]) for d in range(DE)], axis=0)
    bmbd = jnp.kron(eye4, p['edge_b'].reshape(H, H))
    kk = jnp.arange(4 * DE)[None, :, None]
    blk = jnp.arange(4 * H)[None, None, :] // H
    dsel = jnp.arange(DE)[:, None, None]
    smat = (kk == blk * DE + dsel).astype(f32).reshape(DE * 4 * DE, 4 * H)
    wih = p['gru_Wih']
    whh = p['gru_Whh']
    wr, wz, wn = (wih[0:H].T, wih[H:2 * H].T, wih[2 * H:].T)
    ur, uz, un = (whh[0:H].T, whh[H:2 * H].T, whh[2 * H:].T)
    bih = p['gru_bih']
    bhh = p['gru_bhh']
    br = (bih[0:H] + bhh[0:H]).reshape(1, H)
    bz = (bih[H:2 * H] + bhh[H:2 * H]).reshape(1, H)
    bi_n = bih[2 * H:].reshape(1, H)
    bh_n = bhh[2 * H:].reshape(1, H)
    cb = p['conv_b'].reshape(1, H)

    h = x0
    for _ in range(3):
        hsrc = _sc_gather(h, src2d)
        m4 = _msg(hsrc.reshape(ET // 4, 4 * H), e_all.reshape(ET // 4, 4 * DE),
                  w3bd, bmbd, smat)
        parts = _sc_scatter(m4.reshape(ET, H), dst2d, zeros_nt)
        h = _gru(parts[0:NT], parts[NT:2 * NT], h, cb, wr, wz, wn,
                 ur, uz, un, br, bz, bi_n, bh_n)

    gid_all = jnp.stack([r1_gid, r2_gid, pm_gid]).astype(jnp.int32)
    wihT = p['lstm_Wih'].T
    feat = jnp.concatenate(
        [x0.reshape(G, N, H), h.reshape(G, N, H)], axis=2)
    go = _s2s(feat, gid_all.reshape(G, N, 1),
              wihT[0:2 * H], wihT[2 * H:4 * H],
              p['lstm_Whh'].T,
              (p['lstm_bih'] + p['lstm_bhh']).reshape(1, 8 * H),
              p['sp_W'][0:2 * H], p['sp_W'][2 * H:4 * H],
              p['sp_b'].reshape(1, RO), p['sp_a'].reshape(1, 1))

    ge = jnp.transpose(go, (1, 0, 2)).reshape(B, G * RO)
    pos = jnp.asarray(pos_neg_sample).astype(f32)
    pn = jnp.zeros((B, 1), f32) + pos
    xin = jnp.concatenate([labels, ge, pn], axis=1)
    eps = jax.random.normal(jax.random.key(42), (B, LAT), dtype=f32)
    dw0 = p['dec_W0']
    y, mu, lv = _mlp(
        xin, eps, ge, pos.reshape(1, 1),
        p['enc_W0'], p['enc_b0'].reshape(1, PH), p['enc_a0'].reshape(1, 1),
        p['enc_W1'], p['enc_b1'].reshape(1, PH), p['enc_a1'].reshape(1, 1),
        p['enc_W2'], p['enc_b2'].reshape(1, PH), p['enc_a2'].reshape(1, 1),
        p['enc_W3'], p['enc_b3'].reshape(1, 2 * LAT),
        dw0[0:LAT], dw0[LAT:LAT + G * RO], dw0[LAT + G * RO:],
        p['dec_b0'].reshape(1, PH), p['dec_a0'].reshape(1, 1),
        p['dec_W1'], p['dec_b1'].reshape(1, PH), p['dec_a1'].reshape(1, 1),
        p['dec_W2'], p['dec_b2'].reshape(1, PH), p['dec_a2'].reshape(1, 1),
        p['dec_W3'], p['dec_b3'].reshape(1, NCLS))
    return (y, mu, lv)


# speedup vs baseline: 1.2791x; 1.0040x over previous
import functools

import jax
import jax.numpy as jnp
from jax import lax
from jax.experimental import pallas as pl
from jax.experimental.pallas import tpu as pltpu
from jax.experimental.pallas import tpu_sc as plsc

H = 64
DN = 128
DE = 16
RO = 1024
PH = 512
LAT = 128
NCLS = 100
B = 256
N = 8192
E = 16384
G = 3
NT = G * N
ET = G * E

SC_CORES = 2
SC_SUB = 16
NW = SC_CORES * SC_SUB
CH = 128
RW = ET // NW
KC = RW // CH
NZ = NT // SC_SUB

@functools.cache
def _sc_kernels():
    mesh = plsc.VectorSubcoreMesh(
        core_axis_name="c", subcore_axis_name="s",
        num_cores=SC_CORES, num_subcores=SC_SUB)

    @functools.partial(
        pl.kernel,
        out_type=jax.ShapeDtypeStruct((ET, H), jnp.float32),
        mesh=mesh,
        compiler_params=pltpu.CompilerParams(use_tc_tiling_on_sc=False),
        scratch_types=[
            pltpu.VMEM((KC, CH), jnp.int32),
            pltpu.VMEM((RW, H), jnp.float32),
            pltpu.SemaphoreType.DMA,
        ],
    )
    def sc_gather(table_hbm, idx_hbm, out_hbm, idx_v, rows_v, sem):
        c = lax.axis_index("c")
        s = lax.axis_index("s")
        wid = s * SC_CORES + c
        pltpu.sync_copy(idx_hbm.at[wid], idx_v)
        descs = []
        for j in range(KC):
            descs.append(pltpu.async_copy(
                table_hbm.at[idx_v.at[j]], rows_v.at[pl.ds(j * CH, CH)], sem))
        for d in descs:
            d.wait()
        pltpu.sync_copy(rows_v, out_hbm.at[pl.ds(wid * RW, RW)])

    @functools.partial(
        pl.kernel,
        out_type=jax.ShapeDtypeStruct((SC_CORES * NT, H), jnp.float32),
        mesh=mesh,
        compiler_params=pltpu.CompilerParams(use_tc_tiling_on_sc=False),
        scratch_types=[
            pltpu.VMEM((KC, CH), jnp.int32),
            pltpu.VMEM((2, CH, H), jnp.float32),
            pltpu.VMEM_SHARED((NT, H), jnp.float32),
            pltpu.SemaphoreType.DMA,
            pltpu.SemaphoreType.DMA,
        ],
    )
    def sc_scatter(m_hbm, idx_hbm, zeros_hbm, out_hbm, idx_v, m_v, acc_sh,
                   sem0, sem1):
        c = lax.axis_index("c")
        s = lax.axis_index("s")
        zbase = s * NZ
        pltpu.sync_copy(zeros_hbm.at[pl.ds(zbase, NZ)],
                        acc_sh.at[pl.ds(zbase, NZ)])
        ebase = c * (ET // SC_CORES) + s * RW
        pltpu.sync_copy(idx_hbm.at[c * SC_SUB + s], idx_v)
        plsc.subcore_barrier()
        sems = (sem0, sem1)
        descs = [None, None]
        descs[0] = pltpu.async_copy(
            m_hbm.at[pl.ds(ebase, CH)], m_v.at[0], sems[0])
        for j in range(KC):
            cur = j % 2
            nxt = (j + 1) % 2
            if j + 1 < KC:
                descs[nxt] = pltpu.async_copy(
                    m_hbm.at[pl.ds(ebase + (j + 1) * CH, CH)],
                    m_v.at[nxt], sems[nxt])
            descs[cur].wait()
            pltpu.sync_copy(m_v.at[cur], acc_sh.at[idx_v.at[j]], add=True)
        plsc.subcore_barrier()
        pltpu.sync_copy(acc_sh.at[pl.ds(zbase, NZ)],
                        out_hbm.at[pl.ds(c * NT + zbase, NZ)])

    return sc_gather, sc_scatter


def _sc_gather(table, idx2d):
    return _sc_kernels()[0](table, idx2d)


def _sc_scatter(m, idx2d, zeros):
    return _sc_kernels()[1](m, idx2d, zeros)


def _proj_body(x_ref, w_ref, b_ref, o_ref):
    o_ref[...] = jnp.maximum(
        jnp.dot(x_ref[...], w_ref[...], preferred_element_type=jnp.float32)
        + b_ref[...], 0.0)


_proj = pl.pallas_call(
    _proj_body,
    grid=(NT // 2048,),
    in_specs=[pl.BlockSpec((2048, DN), lambda i: (i, 0)),
              pl.BlockSpec((DN, H), lambda i: (0, 0)),
              pl.BlockSpec((1, H), lambda i: (0, 0))],
    out_specs=pl.BlockSpec((2048, H), lambda i: (i, 0)),
    out_shape=jax.ShapeDtypeStruct((NT, H), jnp.float32),
)


def _msg_body(hs4_ref, ea4_ref, w3bd_ref, bmbd_ref, s_ref, o_ref):
    hs4 = hs4_ref[...]
    ea4 = ea4_ref[...]
    acc = jnp.dot(hs4, bmbd_ref[...], preferred_element_type=jnp.float32)
    for d in range(DE):
        scale = jnp.dot(ea4, s_ref[pl.ds(d * H, H), :],
                        preferred_element_type=jnp.float32)
        acc = acc + scale * jnp.dot(
            hs4, w3bd_ref[pl.ds(d * 4 * H, 4 * H), :],
            preferred_element_type=jnp.float32)
    o_ref[...] = acc


_msg = pl.pallas_call(
    _msg_body,
    grid=(ET // 2048,),
    in_specs=[pl.BlockSpec((512, 4 * H), lambda i: (i, 0)),
              pl.BlockSpec((512, 4 * DE), lambda i: (i, 0)),
              pl.BlockSpec((DE * 4 * H, 4 * H), lambda i: (0, 0)),
              pl.BlockSpec((4 * H, 4 * H), lambda i: (0, 0)),
              pl.BlockSpec((DE * H, 4 * H), lambda i: (0, 0))],
    out_specs=pl.BlockSpec((512, 4 * H), lambda i: (i, 0)),
    out_shape=jax.ShapeDtypeStruct((ET // 4, 4 * H), jnp.float32),
)


def _gru_body(p0_ref, p1_ref, h_ref, cb_ref, wr, wz, wn, ur, uz, un,
              br, bz, bi_n, bh_n, o_ref):
    a = jnp.maximum(p0_ref[...] + p1_ref[...] + cb_ref[...], 0.0)
    h = h_ref[...]
    dot = lambda x, w: jnp.dot(x, w[...], preferred_element_type=jnp.float32)
    r = jax.nn.sigmoid(dot(a, wr) + dot(h, ur) + br[...])
    z = jax.nn.sigmoid(dot(a, wz) + dot(h, uz) + bz[...])
    n = jnp.tanh(dot(a, wn) + bi_n[...] + r * (dot(h, un) + bh_n[...]))
    o_ref[...] = (1.0 - z) * n + z * h


_gru = pl.pallas_call(
    _gru_body,
    grid=(NT // 2048,),
    in_specs=[pl.BlockSpec((2048, H), lambda i: (i, 0)),
              pl.BlockSpec((2048, H), lambda i: (i, 0)),
              pl.BlockSpec((2048, H), lambda i: (i, 0)),
              pl.BlockSpec((1, H), lambda i: (0, 0))]
             + [pl.BlockSpec((H, H), lambda i: (0, 0))] * 6
             + [pl.BlockSpec((1, H), lambda i: (0, 0))] * 4,
    out_specs=pl.BlockSpec((2048, H), lambda i: (i, 0)),
    out_shape=jax.ShapeDtypeStruct((NT, H), jnp.float32),
)


CHK = 8192
NCH = N // CHK


def _s2s_body(feat_ref, gidc_ref, wihq, wihr, whh, bb,
              spq, spr, spb, spa, out_ref, mask_ref, ex_ref):
    dot = lambda x, w: jnp.dot(x, w, preferred_element_type=jnp.float32)
    iota_cb = lax.broadcasted_iota(jnp.int32, (CHK, B), 1)
    cdims = (((0,), (0,)), ((), ()))
    bdims = (((1,), (1,)), ((), ()))

    def build(ci, z):
        sl = pl.ds(ci * CHK, CHK)
        mask_ref[sl, :] = (gidc_ref[0, sl, :] == iota_cb).astype(jnp.float32)
        return z

    lax.fori_loop(0, NCH, build, 0)

    q = jnp.zeros((B, 2 * H), jnp.float32)
    rr = jnp.zeros((B, 2 * H), jnp.float32)
    hl = jnp.zeros((B, 2 * H), jnp.float32)
    cl = jnp.zeros((B, 2 * H), jnp.float32)
    for _ in range(3):
        g = (dot(q, wihq[...]) + dot(rr, wihr[...])
             + dot(hl, whh[...]) + bb[...])
        i = jax.nn.sigmoid(g[:, 0:128])
        f = jax.nn.sigmoid(g[:, 128:256])
        gg = jnp.tanh(g[:, 256:384])
        o = jax.nn.sigmoid(g[:, 384:512])
        cl = f * cl + i * gg
        hl = o * jnp.tanh(cl)
        q = hl

        def pass_a(ci, den):
            sl = pl.ds(ci * CHK, CHK)
            mc = mask_ref[sl, :]
            qg = dot(mc, q)
            esc = jnp.sum(feat_ref[0, sl, :] * qg, axis=1, keepdims=True)
            ex = jnp.exp(esc)
            ex_ref[sl, :] = ex
            return den + lax.dot_general(
                ex, mc, cdims, preferred_element_type=jnp.float32)

        den = lax.fori_loop(0, NCH, pass_a, jnp.zeros((1, B), jnp.float32))

        def pass_b(ci, ra):
            sl = pl.ds(ci * CHK, CHK)
            mc = mask_ref[sl, :]
            den_n = lax.dot_general(mc, den, bdims,
                                    preferred_element_type=jnp.float32)
            alpha = ex_ref[sl, :] / den_n
            return ra + lax.dot_general(mc, feat_ref[0, sl, :] * alpha, cdims,
                                        preferred_element_type=jnp.float32)

        rr = lax.fori_loop(0, NCH, pass_b,
                           jnp.zeros((B, 2 * H), jnp.float32))
    out = dot(q, spq[...]) + dot(rr, spr[...]) + spb[...]
    a = spa[0, 0]
    out_ref[0] = jnp.where(out >= 0, out, a * out)


_s2s = pl.pallas_call(
    _s2s_body,
    grid=(G,),
    scratch_shapes=[pltpu.VMEM((N, B), jnp.float32),
                    pltpu.VMEM((N, 1), jnp.float32)],
    in_specs=[pl.BlockSpec((1, N, 2 * H), lambda g: (g, 0, 0)),
              pl.BlockSpec((1, N, 1), lambda g: (g, 0, 0)),
              pl.BlockSpec((2 * H, 512), lambda g: (0, 0)),
              pl.BlockSpec((2 * H, 512), lambda g: (0, 0)),
              pl.BlockSpec((2 * H, 512), lambda g: (0, 0)),
              pl.BlockSpec((1, 512), lambda g: (0, 0)),
              pl.BlockSpec((2 * H, RO), lambda g: (0, 0)),
              pl.BlockSpec((2 * H, RO), lambda g: (0, 0)),
              pl.BlockSpec((1, RO), lambda g: (0, 0)),
              pl.BlockSpec((1, 1), lambda g: (0, 0))],
    out_specs=pl.BlockSpec((1, B, RO), lambda g: (g, 0, 0)),
    out_shape=jax.ShapeDtypeStruct((G, B, RO), jnp.float32),
)


def _mlp_body(xin, eps, ge, pnv,
              w0, b0, a0, w1, b1, a1, w2, b2, a2, w3, b3,
              dw0a, dw0b, dw0c, db0, da0, dw1, db1, da1,
              dw2, db2, da2, dw3, db3,
              y_ref, mu_ref, lv_ref):
    dot = lambda x, w: jnp.dot(x, w[...], preferred_element_type=jnp.float32)
    prelu = lambda x, a: jnp.where(x >= 0, x, a[0, 0] * x)
    x = prelu(dot(xin[...], w0) + b0[...], a0)
    x = prelu(dot(x, w1) + b1[...], a1)
    x = prelu(dot(x, w2) + b2[...], a2)
    x = dot(x, w3) + b3[...]
    mu = jnp.clip(x[:, 0:LAT], -10.0, 10.0)
    lv = jnp.clip(x[:, LAT:2 * LAT], -10.0, 10.0)
    latent = mu + eps[...] * jnp.exp(0.5 * lv)
    y = prelu(dot(latent, dw0a) + dot(ge[...], dw0b)
              + pnv[0, 0] * dw0c[...] + db0[...], da0)
    y = prelu(dot(y, dw1) + db1[...], da1)
    y = prelu(dot(y, dw2) + db2[...], da2)
    y = dot(y, dw3) + db3[...]
    y_ref[...] = jnp.clip(y, -10.0, 10.0)
    mu_ref[...] = mu
    lv_ref[...] = lv


_EI = RO * G + NCLS + 1


def _full(shape):
    return pl.BlockSpec(shape, lambda: tuple(0 for _ in shape))


_mlp = pl.pallas_call(
    _mlp_body,
    in_specs=[_full((B, _EI)), _full((B, LAT)), _full((B, G * RO)),
              _full((1, 1)),
              _full((_EI, PH)), _full((1, PH)), _full((1, 1)),
              _full((PH, PH)), _full((1, PH)), _full((1, 1)),
              _full((PH, PH)), _full((1, PH)), _full((1, 1)),
              _full((PH, 2 * LAT)), _full((1, 2 * LAT)),
              _full((LAT, PH)), _full((G * RO, PH)), _full((1, PH)),
              _full((1, PH)), _full((1, 1)),
              _full((PH, PH)), _full((1, PH)), _full((1, 1)),
              _full((PH, PH)), _full((1, PH)), _full((1, 1)),
              _full((PH, NCLS)), _full((1, NCLS))],
    out_specs=[_full((B, NCLS)), _full((B, LAT)), _full((B, LAT))],
    out_shape=[jax.ShapeDtypeStruct((B, NCLS), jnp.float32),
               jax.ShapeDtypeStruct((B, LAT), jnp.float32),
               jax.ShapeDtypeStruct((B, LAT), jnp.float32)],
)


def kernel(r1_x, r1_e, r1_src, r1_dst, r1_gid, r2_x, r2_e, r2_src, r2_dst,
           r2_gid, pm_x, pm_e, pm_src, pm_dst, pm_gid, labels,
           pos_neg_sample, params):
    p = params
    f32 = jnp.float32
    x_all = jnp.concatenate([r1_x, r2_x, pm_x], axis=0)
    e_all = jnp.concatenate([r1_e, r2_e, pm_e], axis=0)
    src_all = jnp.concatenate(
        [r1_src, r2_src + N, pm_src + 2 * N]).astype(jnp.int32)
    dst_all = jnp.concatenate(
        [r1_dst, r2_dst + N, pm_dst + 2 * N]).astype(jnp.int32)
    src2d = src_all.reshape(NW, KC, CH)
    dst2d = dst_all.reshape(NW, KC, CH)
    zeros_nt = jnp.zeros((NT, H), f32)

    x0 = _proj(x_all, p['proj_W'], p['proj_b'].reshape(1, H))

    eye4 = jnp.eye(4, dtype=f32)
    w3s = p['edge_W'].reshape(DE, H, H)
    w3bd = jnp.concatenate(
        [jnp.kron(eye4, w3s[d]) for d in range(DE)], axis=0)
    bmbd = jnp.kron(eye4, p['edge_b'].reshape(H, H))
    kk = jnp.arange(4 * DE)[None, :, None]
    blk = jnp.arange(4 * H)[None, None, :] // H
    dsel = jnp.arange(DE)[:, None, None]
    smat = (kk == blk * DE + dsel).astype(f32).reshape(DE * 4 * DE, 4 * H)
    wih = p['gru_Wih']
    whh = p['gru_Whh']
    wr, wz, wn = (wih[0:H].T, wih[H:2 * H].T, wih[2 * H:].T)
    ur, uz, un = (whh[0:H].T, whh[H:2 * H].T, whh[2 * H:].T)
    bih = p['gru_bih']
    bhh = p['gru_bhh']
    br = (bih[0:H] + bhh[0:H]).reshape(1, H)
    bz = (bih[H:2 * H] + bhh[H:2 * H]).reshape(1, H)
    bi_n = bih[2 * H:].reshape(1, H)
    bh_n = bhh[2 * H:].reshape(1, H)
    cb = p['conv_b'].reshape(1, H)

    h = x0
    for _ in range(3):
        hsrc = _sc_gather(h, src2d)
        m4 = _msg(hsrc.reshape(ET // 4, 4 * H), e_all.reshape(ET // 4, 4 * DE),
                  w3bd, bmbd, smat)
        parts = _sc_scatter(m4.reshape(ET, H), dst2d, zeros_nt)
        h = _gru(parts[0:NT], parts[NT:2 * NT], h, cb, wr, wz, wn,
                 ur, uz, un, br, bz, bi_n, bh_n)

    gid_all = jnp.stack([r1_gid, r2_gid, pm_gid]).astype(jnp.int32)
    wihT = p['lstm_Wih'].T
    feat = jnp.concatenate(
        [x0.reshape(G, N, H), h.reshape(G, N, H)], axis=2)
    go = _s2s(feat, gid_all.reshape(G, N, 1),
              wihT[0:2 * H], wihT[2 * H:4 * H],
              p['lstm_Whh'].T,
              (p['lstm_bih'] + p['lstm_bhh']).reshape(1, 8 * H),
              p['sp_W'][0:2 * H], p['sp_W'][2 * H:4 * H],
              p['sp_b'].reshape(1, RO), p['sp_a'].reshape(1, 1))

    ge = jnp.transpose(go, (1, 0, 2)).reshape(B, G * RO)
    pos = jnp.asarray(pos_neg_sample).astype(f32)
    pn = jnp.zeros((B, 1), f32) + pos
    xin = jnp.concatenate([labels, ge, pn], axis=1)
    eps = jax.random.normal(jax.random.key(42), (B, LAT), dtype=f32)
    dw0 = p['dec_W0']
    y, mu, lv = _mlp(
        xin, eps, ge, pos.reshape(1, 1),
        p['enc_W0'], p['enc_b0'].reshape(1, PH), p['enc_a0'].reshape(1, 1),
        p['enc_W1'], p['enc_b1'].reshape(1, PH), p['enc_a1'].reshape(1, 1),
        p['enc_W2'], p['enc_b2'].reshape(1, PH), p['enc_a2'].reshape(1, 1),
        p['enc_W3'], p['enc_b3'].reshape(1, 2 * LAT),
        dw0[0:LAT], dw0[LAT:LAT + G * RO], dw0[LAT + G * RO:],
        p['dec_b0'].reshape(1, PH), p['dec_a0'].reshape(1, 1),
        p['dec_W1'], p['dec_b1'].reshape(1, PH), p['dec_a1'].reshape(1, 1),
        p['dec_W2'], p['dec_b2'].reshape(1, PH), p['dec_a2'].reshape(1, 1),
        p['dec_W3'], p['dec_b3'].reshape(1, NCLS))
    return (y, mu, lv)
